# Initial kernel scaffold; baseline (speedup 1.0000x reference)
import jax
import jax.numpy as jnp
from jax.experimental import pallas as pl


def kernel(x, adj, num_graphs, in_batch, cluster, params):
    raise NotImplementedError("write your pallas kernel here")


# speedup vs baseline: 9.9157x; 9.9157x over previous
import functools

import jax
import jax.numpy as jnp
from jax import lax
from jax.experimental import pallas as pl
from jax.experimental.pallas import tpu as pltpu
from jax.experimental.pallas import tpu_sc as plsc

N = 10000
NP = 10240
E = 320000
M = 800
MP = 1024
CN = 100
DUMP = M * M
APLEN = 640256
CHUNK = 80

f32 = jnp.float32
i32 = jnp.int32


def _mesh():
    return plsc.VectorSubcoreMesh(core_axis_name="c", subcore_axis_name="s")


def _fill1d(ref, n, val, dtype):
    def body(i, _):
        ref[pl.ds(i * 16, 16)] = jnp.full((16,), val, dtype)
        return 0
    lax.fori_loop(0, n // 16, body, 0)


def _zero2d(ref, w):
    def body(i, _):
        def inner(j, c):
            ref[i, pl.ds(j * 16, 16)] = jnp.zeros((16,), f32)
            return c
        lax.fori_loop(0, w // 16, inner, 0)
        return 0
    lax.fori_loop(0, 16, body, 0)


def _sc_prep(row, col, bc):
    it = E // 32 // CHUNK

    @functools.partial(
        pl.kernel,
        out_type=(
            jax.ShapeDtypeStruct((NP,), f32),
            jax.ShapeDtypeStruct((NP,), f32),
            jax.ShapeDtypeStruct((APLEN,), f32),
            jax.ShapeDtypeStruct((APLEN,), f32),
        ),
        mesh=_mesh(),
        compiler_params=pltpu.CompilerParams(use_tc_tiling_on_sc=False),
        scratch_types=[
            pltpu.VMEM((CHUNK,), i32),
            pltpu.VMEM((CHUNK,), i32),
            pltpu.VMEM((CHUNK,), i32),
            pltpu.VMEM((CHUNK,), i32),
            pltpu.VMEM((CHUNK,), i32),
            pltpu.VMEM((CHUNK,), f32),
            pltpu.VMEM((4096,), f32),
            pltpu.VMEM_SHARED((NP,), f32),
            pltpu.VMEM_SHARED((APLEN,), f32),
            pltpu.SemaphoreType.DMA,
        ],
    )
    def k(row_h, col_h, bc_h, deg0_h, deg1_h, ap0_h, ap1_h,
          idx_r, idx_c, e0_v, e1_v, apv, ones_v, zline,
          acc_deg, acc_ap, sem):
        c = lax.axis_index("c")
        s = lax.axis_index("s")
        _fill1d(ones_v, CHUNK, 1.0, f32)
        _fill1d(zline, 4096, 0.0, f32)

        pltpu.sync_copy(zline.at[pl.ds(0, NP // 16)],
                        acc_deg.at[pl.ds(s * (NP // 16), NP // 16)])
        span = APLEN // 16
        def zap(j, _):
            pltpu.sync_copy(zline, acc_ap.at[pl.ds(s * span + j * 4096, 4096)])
            return 0
        lax.fori_loop(0, 9, zap, 0)
        pltpu.sync_copy(zline.at[pl.ds(0, 3152)],
                        acc_ap.at[pl.ds(s * span + 9 * 4096, 3152)])
        plsc.subcore_barrier()

        base = c * (E // 2) + s * (E // 32)

        def step(j, _):
            b = base + j * CHUNK
            pltpu.sync_copy(row_h.at[pl.ds(b, CHUNK)], idx_r)
            pltpu.sync_copy(col_h.at[pl.ds(b, CHUNK)], idx_c)
            pltpu.sync_copy(ones_v, acc_deg.at[idx_c], add=True)
            pltpu.async_copy(bc_h.at[idx_r], e0_v, sem).wait()
            pltpu.async_copy(bc_h.at[idx_c], e1_v, sem).wait()

            def grp(g, _):
                e0 = e0_v[pl.ds(g * 16, 16)]
                e1 = e1_v[pl.ds(g * 16, 16)]
                pid = jnp.where(e0 != e1, e1 * M + e0, DUMP)
                apv[pl.ds(g * 16, 16)] = pid
                return 0
            lax.fori_loop(0, CHUNK // 16, grp, 0)
            pltpu.sync_copy(ones_v, acc_ap.at[apv], add=True)
            return 0
        lax.fori_loop(0, it, step, 0)
        plsc.subcore_barrier()

        rw = NP // 16

        def wout(deg_h, ap_h):
            pltpu.sync_copy(acc_deg.at[pl.ds(s * rw, rw)],
                            zline.at[pl.ds(0, rw)])
            pltpu.sync_copy(zline.at[pl.ds(0, rw)],
                            deg_h.at[pl.ds(s * rw, rw)])

            def wr(j, _):
                pltpu.sync_copy(acc_ap.at[pl.ds(s * span + j * 4096, 4096)],
                                zline)
                pltpu.sync_copy(zline,
                                ap_h.at[pl.ds(s * span + j * 4096, 4096)])
                return 0
            lax.fori_loop(0, 9, wr, 0)
            pltpu.sync_copy(acc_ap.at[pl.ds(s * span + 9 * 4096, 3152)],
                            zline.at[pl.ds(0, 3152)])
            pltpu.sync_copy(zline.at[pl.ds(0, 3152)],
                            ap_h.at[pl.ds(s * span + 9 * 4096, 3152)])

        @pl.when(c == 0)
        def _():
            wout(deg0_h, ap0_h)

        @pl.when(c == 1)
        def _():
            wout(deg1_h, ap1_h)

    return k(row, col, bc)


def _sc_agg2(W, hp0, hp1, row, col):
    it = E // 16 // CHUNK

    @functools.partial(
        pl.kernel,
        out_type=(
            jax.ShapeDtypeStruct((NP, W), f32),
            jax.ShapeDtypeStruct((NP, W), f32),
        ),
        mesh=_mesh(),
        compiler_params=pltpu.CompilerParams(use_tc_tiling_on_sc=False),
        scratch_types=[
            pltpu.VMEM((CHUNK,), i32),
            pltpu.VMEM((CHUNK,), i32),
            pltpu.VMEM((CHUNK, W), f32),
            pltpu.VMEM((16, W), f32),
            pltpu.VMEM_SHARED((NP, W), f32),
            pltpu.SemaphoreType.DMA,
        ],
    )
    def k(row_h, col_h, hp0_h, hp1_h, s0_h, s1_h,
          idx_r, idx_c, rows_v, zrows, acc, sem):
        c = lax.axis_index("c")
        s = lax.axis_index("s")
        _zero2d(zrows, W)
        rpt = NP // 16

        def zb(i, _):
            pltpu.sync_copy(zrows, acc.at[pl.ds(s * rpt + i * 16, 16), :])
            return 0
        lax.fori_loop(0, rpt // 16, zb, 0)
        plsc.subcore_barrier()

        def work(hp_h):
            base = s * (E // 16)

            def step(j, _):
                b = base + j * CHUNK
                pltpu.sync_copy(row_h.at[pl.ds(b, CHUNK)], idx_r)
                pltpu.sync_copy(col_h.at[pl.ds(b, CHUNK)], idx_c)
                pltpu.async_copy(hp_h.at[idx_r], rows_v, sem).wait()
                pltpu.sync_copy(rows_v, acc.at[idx_c], add=True)
                return 0
            lax.fori_loop(0, it, step, 0)

        @pl.when(c == 0)
        def _():
            work(hp0_h)

        @pl.when(c == 1)
        def _():
            work(hp1_h)
        plsc.subcore_barrier()

        def wout(out_h):
            def w(i, _):
                r0 = s * rpt + i * CHUNK
                pltpu.sync_copy(acc.at[pl.ds(r0, CHUNK), :], rows_v)
                pltpu.sync_copy(rows_v, out_h.at[pl.ds(r0, CHUNK), :])
                return 0
            lax.fori_loop(0, rpt // CHUNK, w, 0)

        @pl.when(c == 0)
        def _():
            wout(s0_h)

        @pl.when(c == 1)
        def _():
            wout(s1_h)

    return k(row, col, hp0, hp1)


def _sc_agg_eb(W, hp, row, col):
    it = E // 32 // CHUNK

    @functools.partial(
        pl.kernel,
        out_type=(
            jax.ShapeDtypeStruct((NP, W), f32),
            jax.ShapeDtypeStruct((NP, W), f32),
        ),
        mesh=_mesh(),
        compiler_params=pltpu.CompilerParams(use_tc_tiling_on_sc=False),
        scratch_types=[
            pltpu.VMEM((CHUNK,), i32),
            pltpu.VMEM((CHUNK,), i32),
            pltpu.VMEM((CHUNK, W), f32),
            pltpu.VMEM((16, W), f32),
            pltpu.VMEM_SHARED((NP, W), f32),
            pltpu.SemaphoreType.DMA,
        ],
    )
    def k(row_h, col_h, hp_h, s0_h, s1_h,
          idx_r, idx_c, rows_v, zrows, acc, sem):
        c = lax.axis_index("c")
        s = lax.axis_index("s")
        _zero2d(zrows, W)
        rpt = NP // 16

        def zb(i, _):
            pltpu.sync_copy(zrows, acc.at[pl.ds(s * rpt + i * 16, 16), :])
            return 0
        lax.fori_loop(0, rpt // 16, zb, 0)
        plsc.subcore_barrier()

        base = c * (E // 2) + s * (E // 32)

        def step(j, _):
            b = base + j * CHUNK
            pltpu.sync_copy(row_h.at[pl.ds(b, CHUNK)], idx_r)
            pltpu.sync_copy(col_h.at[pl.ds(b, CHUNK)], idx_c)
            pltpu.async_copy(hp_h.at[idx_r], rows_v, sem).wait()
            pltpu.sync_copy(rows_v, acc.at[idx_c], add=True)
            return 0
        lax.fori_loop(0, it, step, 0)
        plsc.subcore_barrier()

        def wout(out_h):
            def w(i, _):
                r0 = s * rpt + i * CHUNK
                pltpu.sync_copy(acc.at[pl.ds(r0, CHUNK), :], rows_v)
                pltpu.sync_copy(rows_v, out_h.at[pl.ds(r0, CHUNK), :])
                return 0
            lax.fori_loop(0, rpt // CHUNK, w, 0)

        @pl.when(c == 0)
        def _():
            wout(s0_h)

        @pl.when(c == 1)
        def _():
            wout(s1_h)

    return k(row, col, hp)


def _sc_pool(vals, idx_p, W):
    rpt = NP // 32

    @functools.partial(
        pl.kernel,
        out_type=(
            jax.ShapeDtypeStruct((MP, W), f32),
            jax.ShapeDtypeStruct((MP, W), f32),
        ),
        mesh=_mesh(),
        compiler_params=pltpu.CompilerParams(use_tc_tiling_on_sc=False),
        scratch_types=[
            pltpu.VMEM((CHUNK,), i32),
            pltpu.VMEM((CHUNK, W), f32),
            pltpu.VMEM((16, W), f32),
            pltpu.VMEM_SHARED((MP, W), f32),
        ],
    )
    def k(vals_h, idx_h, p0_h, p1_h, idxb, rows_v, zrows, acc):
        c = lax.axis_index("c")
        s = lax.axis_index("s")
        _zero2d(zrows, W)
        mpt = MP // 16

        def zb(i, _):
            pltpu.sync_copy(zrows, acc.at[pl.ds(s * mpt + i * 16, 16), :])
            return 0
        lax.fori_loop(0, mpt // 16, zb, 0)
        plsc.subcore_barrier()

        base = (c * 16 + s) * rpt

        def step(j, _):
            b = base + j * CHUNK
            pltpu.sync_copy(idx_h.at[pl.ds(b, CHUNK)], idxb)
            pltpu.sync_copy(vals_h.at[pl.ds(b, CHUNK), :], rows_v)
            pltpu.sync_copy(rows_v, acc.at[idxb], add=True)
            return 0
        lax.fori_loop(0, rpt // CHUNK, step, 0)
        plsc.subcore_barrier()

        def wout(out_h):
            pltpu.sync_copy(acc.at[pl.ds(s * mpt, mpt), :],
                            rows_v.at[pl.ds(0, mpt), :])
            pltpu.sync_copy(rows_v.at[pl.ds(0, mpt), :],
                            out_h.at[pl.ds(s * mpt, mpt), :])

        @pl.when(c == 0)
        def _():
            wout(p0_h)

        @pl.when(c == 1)
        def _():
            wout(p1_h)

    return k(vals, idx_p)


def _sc_gather(table, idx_p, W):
    rpt = NP // 32

    @functools.partial(
        pl.kernel,
        out_type=jax.ShapeDtypeStruct((NP, W), f32),
        mesh=_mesh(),
        compiler_params=pltpu.CompilerParams(use_tc_tiling_on_sc=False),
        scratch_types=[
            pltpu.VMEM((CHUNK,), i32),
            pltpu.VMEM((CHUNK, W), f32),
            pltpu.SemaphoreType.DMA,
        ],
    )
    def k(tab_h, idx_h, out_h, idxb, rows_v, sem):
        c = lax.axis_index("c")
        s = lax.axis_index("s")
        base = (c * 16 + s) * rpt

        def step(j, _):
            b = base + j * CHUNK
            pltpu.sync_copy(idx_h.at[pl.ds(b, CHUNK)], idxb)
            pltpu.async_copy(tab_h.at[idxb], rows_v, sem).wait()
            pltpu.sync_copy(rows_v, out_h.at[pl.ds(b, CHUNK), :])
            return 0
        lax.fori_loop(0, rpt // CHUNK, step, 0)

    return k(table, idx_p)


def _elu(v):
    return jnp.where(v > 0, v, jnp.exp(v) - 1.0)


def _dot(a, b):
    return jnp.dot(a, b, preferred_element_type=f32)


def _tc(fn, out_shape, *args):
    return pl.pallas_call(fn, out_shape=out_shape)(*args)


def _tc_pre(deg0, deg1, ap0_2d, ap1_2d):
    def body(d0, d1, a0, a1, dinv_r, pres_r, dp_r):
        deg = d0[...] + d1[...] + 1.0
        dinv_r[...] = lax.rsqrt(deg)
        pres = ((a0[...] + a1[...]) > 0).astype(f32)
        deg_p = jnp.sum(pres, axis=1, keepdims=True) + 1.0
        pres_r[...] = pres
        dp_r[...] = lax.rsqrt(deg_p)
    return _tc(body, (jax.ShapeDtypeStruct((NP,), f32),
                      jax.ShapeDtypeStruct((M, M), f32),
                      jax.ShapeDtypeStruct((M, 1), f32)),
               deg0, deg1, ap0_2d, ap1_2d)


def _tc0(x, Wg1, dinv_c):
    def body(x_r, w_r, dv_r, o0_r, o1_r):
        h = dv_r[...] * _dot(x_r[...], w_r[...])
        o0_r[...] = h[:, :32]
        o1_r[...] = h[:, 32:]
    return _tc(body, (jax.ShapeDtypeStruct((N, 32), f32),
                      jax.ShapeDtypeStruct((N, 32), f32)),
               x, Wg1, dinv_c)


def _tc1(s0, s1, hp0, hp1, dinv_c, bg1, fw1, fb1, Wg2):
    def body(s0_r, s1_r, h0_r, h1_r, dv_r, b_r, fw_r, fb_r, w2_r, o0_r, o1_r):
        S = jnp.concatenate([s0_r[...][:N], s1_r[...][:N]], axis=1)
        hp = jnp.concatenate([h0_r[...], h1_r[...]], axis=1)
        dv = dv_r[...]
        h1 = _elu(dv * (S + hp) + b_r[...])
        u1 = _elu(_dot(h1, fw_r[...]) + fb_r[...])
        h2p = dv * _dot(u1, w2_r[...])
        o0_r[...] = h2p[:, :128]
        o1_r[...] = h2p[:, 128:]
    return _tc(body, (jax.ShapeDtypeStruct((N, 128), f32),
                      jax.ShapeDtypeStruct((N, 128), f32)),
               s0, s1, hp0, hp1, dinv_c, bg1, fw1, fb1, Wg2)


def _tc2a(s0, s1, hp0, hp1, dinv_c, bg2, fw2, fb2):
    def body(s0_r, s1_r, h0_r, h1_r, dv_r, b_r, fw_r, fb_r, u_r):
        S = jnp.concatenate([s0_r[...][:N], s1_r[...][:N]], axis=1)
        hp = jnp.concatenate([h0_r[...], h1_r[...]], axis=1)
        h2 = _elu(dv_r[...] * (S + hp) + b_r[...])
        u_r[...] = _elu(_dot(h2, fw_r[...]) + fb_r[...])
    return _tc(body, jax.ShapeDtypeStruct((N, 256), f32),
               s0, s1, hp0, hp1, dinv_c, bg2, fw2, fb2)


def _tc2b(u2):
    def body(u_r, o_r):
        u = u_r[...]
        s1 = jnp.sum(u, axis=0, keepdims=True)
        s2 = jnp.sum(u * u, axis=0, keepdims=True)
        mu = s1 / N
        var = s2 / N - mu * mu
        y = (u - mu) * lax.rsqrt(var + 1e-5)
        aug = jnp.concatenate(
            [y, jnp.ones((N, 1), f32), jnp.zeros((N, 15), f32)], axis=1)
        o_r[...] = jnp.concatenate(
            [aug, jnp.zeros((NP - N, 272), f32)], axis=0)
    return _tc(body, jax.ShapeDtypeStruct((NP, 272), f32), u2)


def _tc3(ps0, ps1, y_aug, pres, dp, Wl1, bl1, fwl1, fbl1, Wl2, bl2,
         fwl2, fbl2, Wm1):
    def body(p0_r, p1_r, y_r, pr_r, dp_r, wl1_r, bl1_r, fw1_r, fb1_r,
             wl2_r, bl2_r, fw2_r, fb2_r, wm1_r, g_r, p_out_r, den_r):
        sums = p0_r[...][:M] + p1_r[...][:M]
        denom = jnp.maximum(sums[:, 256:257], 1.0)
        pooled = sums[:, :256] / denom
        pr = pr_r[...]
        dp = dp_r[...]

        def pconv(g, w, b):
            td = dp * _dot(g, w)
            return dp * (_dot(pr, td) + td) + b

        lx = _elu(pconv(pooled, wl1_r[...], bl1_r[...]))
        lx = _elu(_dot(lx, fw1_r[...]) + fb1_r[...])
        lx = _elu(pconv(lx, wl2_r[...], bl2_r[...]))
        lx = _elu(_dot(lx, fw2_r[...]) + fb2_r[...])
        wm1 = wm1_r[...]
        g_r[...] = _dot(lx, wm1[256:])
        p_out_r[...] = _dot(y_r[...][:N, :256], wm1[:256])
        den_r[...] = denom
    return _tc(body, (jax.ShapeDtypeStruct((M, 128), f32),
                      jax.ShapeDtypeStruct((N, 128), f32),
                      jax.ShapeDtypeStruct((M, 1), f32)),
               ps0, ps1, y_aug, pres, dp, Wl1, bl1, fwl1, fbl1,
               Wl2, bl2, fwl2, fbl2, Wm1)


def _tc4(gbc, p_mat, dinv_c):
    def body(g_r, p_r, dv_r, o0_r, o1_r):
        hp3 = dv_r[...] * (p_r[...] + g_r[...][:N])
        o0_r[...] = hp3[:, :64]
        o1_r[...] = hp3[:, 64:]
    return _tc(body, (jax.ShapeDtypeStruct((N, 64), f32),
                      jax.ShapeDtypeStruct((N, 64), f32)),
               gbc, p_mat, dinv_c)


def _tc5(s0, s1, hp0, hp1, dinv_c, bm1, fwm1, fbm1, Wm2):
    def body(s0_r, s1_r, h0_r, h1_r, dv_r, b_r, fw_r, fb_r, w2_r, o0_r, o1_r):
        S = jnp.concatenate([s0_r[...][:N], s1_r[...][:N]], axis=1)
        hp = jnp.concatenate([h0_r[...], h1_r[...]], axis=1)
        dv = dv_r[...]
        z1 = _elu(dv * (S + hp) + b_r[...])
        u = _elu(_dot(z1, fw_r[...]) + fb_r[...])
        hp4 = dv * _dot(u, w2_r[...])
        o0_r[...] = hp4[:, :16]
        o1_r[...] = hp4[:, 16:]
    return _tc(body, (jax.ShapeDtypeStruct((N, 16), f32),
                      jax.ShapeDtypeStruct((N, 16), f32)),
               s0, s1, hp0, hp1, dinv_c, bm1, fwm1, fbm1, Wm2)


def _tc6(s0, s1, hp0, hp1, dinv_c, bm2, fwm2, fbm2, W3p):
    def body(s0_r, s1_r, h0_r, h1_r, dv_r, b_r, fw_r, fb_r, w3_r, o_r):
        S = jnp.concatenate([s0_r[...][:N], s1_r[...][:N]], axis=1)
        hp = jnp.concatenate([h0_r[...], h1_r[...]], axis=1)
        dv = dv_r[...]
        h = _elu(dv * (S + hp) + b_r[...])
        u = _elu(_dot(h, fw_r[...]) + fb_r[...])
        o_r[...] = dv * _dot(u, w3_r[...])
    return _tc(body, jax.ShapeDtypeStruct((N, 16), f32),
               s0, s1, hp0, hp1, dinv_c, bm2, fwm2, fbm2, W3p)


def _tc7(s0, s1, hp5, dinv_c, b3p, fw3p, fb3p):
    def body(s0_r, s1_r, h_r, dv_r, b_r, fw_r, fb_r, o_r):
        S = s0_r[...][:N] + s1_r[...][:N]
        z = _elu(dv_r[...] * (S + h_r[...]) + b_r[...])
        z = _elu(_dot(z, fw_r[...]) + fb_r[...])
        o_r[...] = jnp.concatenate([z, jnp.zeros((NP - N, 16), f32)], axis=0)
    return _tc(body, jax.ShapeDtypeStruct((NP, 16), f32),
               s0, s1, hp5, dinv_c, b3p, fw3p, fb3p)


def _tc8(pz0, pz1, denom, pres, dp, WOp, bOp, fwOp, fbOp):
    def body(p0_r, p1_r, den_r, pr_r, dp_r, wo_r, bo_r, fw_r, fb_r, o_r):
        pooled = (p0_r[...][:M] + p1_r[...][:M]) / den_r[...]
        pr = pr_r[...]
        dp = dp_r[...]
        td = dp * _dot(pooled, wo_r[...])
        zz = _elu(dp * (_dot(pr, td) + td) + bo_r[...])
        o_r[...] = _elu(_dot(zz, fw_r[...]) + fb_r[...])
    return _tc(body, jax.ShapeDtypeStruct((M, 16), f32),
               pz0, pz1, denom, pres, dp, WOp, bOp, fwOp, fbOp)


def kernel(x, adj, num_graphs, in_batch, cluster, params):
    p = params
    row, col = adj[0], adj[1]

    W3p = jnp.zeros((32, 16), f32).at[:, :3].set(p["GCN_M3_W"])
    b3p = jnp.zeros((16,), f32).at[:3].set(p["GCN_M3_b"])
    fw3p = jnp.zeros((16, 16), f32).at[:3, :3].set(p["fc_M3_W"])
    fb3p = jnp.zeros((16,), f32).at[:3].set(p["fc_M3_b"])
    WOp = jnp.zeros((16, 16), f32).at[:3, :3].set(p["GCN_O1_W"])
    bOp = jnp.zeros((16,), f32).at[:3].set(p["GCN_O1_b"])
    fwOp = jnp.zeros((16, 16), f32).at[:3, :3].set(p["fc_O1_W"])
    fbOp = jnp.zeros((16,), f32).at[:3].set(p["fc_O1_b"])

    bc = cluster + in_batch * CN
    bc_p = jnp.concatenate([bc, jnp.zeros(NP - N, i32)])
    deg0, deg1, ap0, ap1 = _sc_prep(row, col, bc)
    ap0_2d = ap0[: M * M].reshape(M, M)
    ap1_2d = ap1[: M * M].reshape(M, M)
    dinv1d, pres, dp = _tc_pre(deg0, deg1, ap0_2d, ap1_2d)
    dinv_c = dinv1d[:N].reshape(N, 1)

    hp1a, hp1b = _tc0(x, p["GCN_G1_W"], dinv_c)
    s1a, s1b = _sc_agg2(32, hp1a, hp1b, row, col)
    hp2a, hp2b = _tc1(s1a, s1b, hp1a, hp1b, dinv_c, p["GCN_G1_b"],
                      p["fc_G1_W"], p["fc_G1_b"], p["GCN_G2_W"])
    s2a, s2b = _sc_agg2(128, hp2a, hp2b, row, col)
    u2 = _tc2a(s2a, s2b, hp2a, hp2b, dinv_c, p["GCN_G2_b"],
               p["fc_G2_W"], p["fc_G2_b"])
    y_aug = _tc2b(u2)

    ps0, ps1 = _sc_pool(y_aug, bc_p, 272)
    G, P, denom = _tc3(ps0, ps1, y_aug, pres, dp,
                       p["GCN_L1_W"], p["GCN_L1_b"], p["fc_L1_W"], p["fc_L1_b"],
                       p["GCN_L2_W"], p["GCN_L2_b"], p["fc_L2_W"], p["fc_L2_b"],
                       p["GCN_M1_W"])
    Gbc = _sc_gather(G, bc_p, 128)
    hp3a, hp3b = _tc4(Gbc, P, dinv_c)
    s3a, s3b = _sc_agg2(64, hp3a, hp3b, row, col)
    hp4a, hp4b = _tc5(s3a, s3b, hp3a, hp3b, dinv_c, p["GCN_M1_b"],
                      p["fc_M1_W"], p["fc_M1_b"], p["GCN_M2_W"])
    s4a, s4b = _sc_agg2(16, hp4a, hp4b, row, col)
    hp5 = _tc6(s4a, s4b, hp4a, hp4b, dinv_c, p["GCN_M2_b"],
               p["fc_M2_W"], p["fc_M2_b"], W3p)
    s5a, s5b = _sc_agg_eb(16, hp5, row, col)
    z8 = _tc7(s5a, s5b, hp5, dinv_c, b3p, fw3p, fb3p)

    pz0, pz1 = _sc_pool(z8, bc_p, 16)
    zz8 = _tc8(pz0, pz1, denom, pres, dp, WOp, bOp, fwOp, fbOp)
    orows = _sc_gather(zz8, bc_p, 16)
    return (orows[:N, :3], zz8[:, :3])


# speedup vs baseline: 14.3332x; 1.4455x over previous
import functools

import jax
import jax.numpy as jnp
from jax import lax
from jax.experimental import pallas as pl
from jax.experimental.pallas import tpu as pltpu
from jax.experimental.pallas import tpu_sc as plsc

N = 10000
NP = 10240
E = 320000
M = 800
MP = 1024
CN = 100
DUMP = M * M
APLEN = 640256
CHUNK = 80

f32 = jnp.float32
i32 = jnp.int32


def _mesh():
    return plsc.VectorSubcoreMesh(core_axis_name="c", subcore_axis_name="s")


def _fill1d(ref, n, val, dtype):
    def body(i, _):
        ref[pl.ds(i * 16, 16)] = jnp.full((16,), val, dtype)
        return 0
    lax.fori_loop(0, n // 16, body, 0)


def _zero2d(ref, w):
    def body(i, _):
        def inner(j, c):
            ref[i, pl.ds(j * 16, 16)] = jnp.zeros((16,), f32)
            return c
        lax.fori_loop(0, w // 16, inner, 0)
        return 0
    lax.fori_loop(0, 16, body, 0)


def _sc_prep(row, col, bc):
    it = E // 32 // CHUNK

    @functools.partial(
        pl.kernel,
        out_type=(
            jax.ShapeDtypeStruct((NP,), f32),
            jax.ShapeDtypeStruct((NP,), f32),
            jax.ShapeDtypeStruct((APLEN,), f32),
            jax.ShapeDtypeStruct((APLEN,), f32),
        ),
        mesh=_mesh(),
        compiler_params=pltpu.CompilerParams(use_tc_tiling_on_sc=False),
        scratch_types=[
            pltpu.VMEM((CHUNK,), i32),
            pltpu.VMEM((CHUNK,), i32),
            pltpu.VMEM((CHUNK,), i32),
            pltpu.VMEM((CHUNK,), i32),
            pltpu.VMEM((CHUNK,), i32),
            pltpu.VMEM((CHUNK,), f32),
            pltpu.VMEM((4096,), f32),
            pltpu.VMEM_SHARED((NP,), f32),
            pltpu.VMEM_SHARED((APLEN,), f32),
            pltpu.SemaphoreType.DMA,
        ],
    )
    def k(row_h, col_h, bc_h, deg0_h, deg1_h, ap0_h, ap1_h,
          idx_r, idx_c, e0_v, e1_v, apv, ones_v, zline,
          acc_deg, acc_ap, sem):
        c = lax.axis_index("c")
        s = lax.axis_index("s")
        _fill1d(ones_v, CHUNK, 1.0, f32)
        _fill1d(zline, 4096, 0.0, f32)

        pltpu.sync_copy(zline.at[pl.ds(0, NP // 16)],
                        acc_deg.at[pl.ds(s * (NP // 16), NP // 16)])
        span = APLEN // 16
        def zap(j, _):
            pltpu.sync_copy(zline, acc_ap.at[pl.ds(s * span + j * 4096, 4096)])
            return 0
        lax.fori_loop(0, 9, zap, 0)
        pltpu.sync_copy(zline.at[pl.ds(0, 3152)],
                        acc_ap.at[pl.ds(s * span + 9 * 4096, 3152)])
        plsc.subcore_barrier()

        base = c * (E // 2) + s * (E // 32)

        def step(j, _):
            b = base + j * CHUNK
            pltpu.sync_copy(row_h.at[pl.ds(b, CHUNK)], idx_r)
            pltpu.sync_copy(col_h.at[pl.ds(b, CHUNK)], idx_c)
            pltpu.sync_copy(ones_v, acc_deg.at[idx_c], add=True)
            pltpu.async_copy(bc_h.at[idx_r], e0_v, sem).wait()
            pltpu.async_copy(bc_h.at[idx_c], e1_v, sem).wait()

            def grp(g, _):
                e0 = e0_v[pl.ds(g * 16, 16)]
                e1 = e1_v[pl.ds(g * 16, 16)]
                pid = jnp.where(e0 != e1, e1 * M + e0, DUMP)
                apv[pl.ds(g * 16, 16)] = pid
                return 0
            lax.fori_loop(0, CHUNK // 16, grp, 0)
            pltpu.sync_copy(ones_v, acc_ap.at[apv], add=True)
            return 0
        lax.fori_loop(0, it, step, 0)
        plsc.subcore_barrier()

        rw = NP // 16

        def wout(deg_h, ap_h):
            pltpu.sync_copy(acc_deg.at[pl.ds(s * rw, rw)],
                            zline.at[pl.ds(0, rw)])
            pltpu.sync_copy(zline.at[pl.ds(0, rw)],
                            deg_h.at[pl.ds(s * rw, rw)])

            def wr(j, _):
                pltpu.sync_copy(acc_ap.at[pl.ds(s * span + j * 4096, 4096)],
                                zline)
                pltpu.sync_copy(zline,
                                ap_h.at[pl.ds(s * span + j * 4096, 4096)])
                return 0
            lax.fori_loop(0, 9, wr, 0)
            pltpu.sync_copy(acc_ap.at[pl.ds(s * span + 9 * 4096, 3152)],
                            zline.at[pl.ds(0, 3152)])
            pltpu.sync_copy(zline.at[pl.ds(0, 3152)],
                            ap_h.at[pl.ds(s * span + 9 * 4096, 3152)])

        @pl.when(c == 0)
        def _():
            wout(deg0_h, ap0_h)

        @pl.when(c == 1)
        def _():
            wout(deg1_h, ap1_h)

    return k(row, col, bc)


def _edge_pipeline(row_h, col_h, hp_h, acc,
                   idx_r0, idx_c0, rows0, idx_r1, idx_c1, rows1,
                   sem0, sem1, base, it):
    def load(j, idx_r, idx_c):
        b = base + j * CHUNK
        pltpu.sync_copy(row_h.at[pl.ds(b, CHUNK)], idx_r)
        pltpu.sync_copy(col_h.at[pl.ds(b, CHUNK)], idx_c)

    load(0, idx_r0, idx_c0)
    pltpu.async_copy(hp_h.at[idx_r0], rows0, sem0)

    def pair(j2, _):
        jA = j2 * 2

        @pl.when(jA + 1 < it)
        def _():
            load(jA + 1, idx_r1, idx_c1)
            pltpu.async_copy(hp_h.at[idx_r1], rows1, sem1)

        pltpu.make_async_copy(hp_h.at[idx_r0], rows0, sem0).wait()
        pltpu.sync_copy(rows0, acc.at[idx_c0], add=True)

        @pl.when(jA + 2 < it)
        def _():
            load(jA + 2, idx_r0, idx_c0)
            pltpu.async_copy(hp_h.at[idx_r0], rows0, sem0)

        @pl.when(jA + 1 < it)
        def _():
            pltpu.make_async_copy(hp_h.at[idx_r1], rows1, sem1).wait()
            pltpu.sync_copy(rows1, acc.at[idx_c1], add=True)
        return 0
    lax.fori_loop(0, (it + 1) // 2, pair, 0)


def _sc_agg2(W, hp0, hp1, row, col):
    it = E // 16 // CHUNK

    @functools.partial(
        pl.kernel,
        out_type=(
            jax.ShapeDtypeStruct((NP, W), f32),
            jax.ShapeDtypeStruct((NP, W), f32),
        ),
        mesh=_mesh(),
        compiler_params=pltpu.CompilerParams(use_tc_tiling_on_sc=False),
        scratch_types=[
            pltpu.VMEM((CHUNK,), i32),
            pltpu.VMEM((CHUNK,), i32),
            pltpu.VMEM((CHUNK, W), f32),
            pltpu.VMEM((CHUNK,), i32),
            pltpu.VMEM((CHUNK,), i32),
            pltpu.VMEM((CHUNK, W), f32),
            pltpu.VMEM((16, W), f32),
            pltpu.VMEM_SHARED((NP, W), f32),
            pltpu.SemaphoreType.DMA,
            pltpu.SemaphoreType.DMA,
        ],
    )
    def k(row_h, col_h, hp0_h, hp1_h, s0_h, s1_h,
          idx_r0, idx_c0, rows0, idx_r1, idx_c1, rows1, zrows, acc,
          sem0, sem1):
        c = lax.axis_index("c")
        s = lax.axis_index("s")
        _zero2d(zrows, W)
        rpt = NP // 16

        def zb(i, _):
            pltpu.sync_copy(zrows, acc.at[pl.ds(s * rpt + i * 16, 16), :])
            return 0
        lax.fori_loop(0, rpt // 16, zb, 0)
        plsc.subcore_barrier()

        def work(hp_h):
            _edge_pipeline(row_h, col_h, hp_h, acc,
                           idx_r0, idx_c0, rows0, idx_r1, idx_c1, rows1,
                           sem0, sem1, s * (E // 16), it)

        @pl.when(c == 0)
        def _():
            work(hp0_h)

        @pl.when(c == 1)
        def _():
            work(hp1_h)
        plsc.subcore_barrier()

        def wout(out_h):
            def w(i, _):
                r0 = s * rpt + i * CHUNK
                pltpu.sync_copy(acc.at[pl.ds(r0, CHUNK), :], rows0)
                pltpu.sync_copy(rows0, out_h.at[pl.ds(r0, CHUNK), :])
                return 0
            lax.fori_loop(0, rpt // CHUNK, w, 0)

        @pl.when(c == 0)
        def _():
            wout(s0_h)

        @pl.when(c == 1)
        def _():
            wout(s1_h)

    return k(row, col, hp0, hp1)


def _sc_agg_eb(W, hp, row, col):
    it = E // 32 // CHUNK

    @functools.partial(
        pl.kernel,
        out_type=(
            jax.ShapeDtypeStruct((NP, W), f32),
            jax.ShapeDtypeStruct((NP, W), f32),
        ),
        mesh=_mesh(),
        compiler_params=pltpu.CompilerParams(use_tc_tiling_on_sc=False),
        scratch_types=[
            pltpu.VMEM((CHUNK,), i32),
            pltpu.VMEM((CHUNK,), i32),
            pltpu.VMEM((CHUNK, W), f32),
            pltpu.VMEM((CHUNK,), i32),
            pltpu.VMEM((CHUNK,), i32),
            pltpu.VMEM((CHUNK, W), f32),
            pltpu.VMEM((16, W), f32),
            pltpu.VMEM_SHARED((NP, W), f32),
            pltpu.SemaphoreType.DMA,
            pltpu.SemaphoreType.DMA,
        ],
    )
    def k(row_h, col_h, hp_h, s0_h, s1_h,
          idx_r0, idx_c0, rows0, idx_r1, idx_c1, rows1, zrows, acc,
          sem0, sem1):
        c = lax.axis_index("c")
        s = lax.axis_index("s")
        _zero2d(zrows, W)
        rpt = NP // 16

        def zb(i, _):
            pltpu.sync_copy(zrows, acc.at[pl.ds(s * rpt + i * 16, 16), :])
            return 0
        lax.fori_loop(0, rpt // 16, zb, 0)
        plsc.subcore_barrier()

        _edge_pipeline(row_h, col_h, hp_h, acc,
                       idx_r0, idx_c0, rows0, idx_r1, idx_c1, rows1,
                       sem0, sem1, c * (E // 2) + s * (E // 32), it)
        plsc.subcore_barrier()

        def wout(out_h):
            def w(i, _):
                r0 = s * rpt + i * CHUNK
                pltpu.sync_copy(acc.at[pl.ds(r0, CHUNK), :], rows0)
                pltpu.sync_copy(rows0, out_h.at[pl.ds(r0, CHUNK), :])
                return 0
            lax.fori_loop(0, rpt // CHUNK, w, 0)

        @pl.when(c == 0)
        def _():
            wout(s0_h)

        @pl.when(c == 1)
        def _():
            wout(s1_h)

    return k(row, col, hp)


def _sc_pool(vals, idx_p, W):
    rpt = NP // 32

    @functools.partial(
        pl.kernel,
        out_type=(
            jax.ShapeDtypeStruct((MP, W), f32),
            jax.ShapeDtypeStruct((MP, W), f32),
        ),
        mesh=_mesh(),
        compiler_params=pltpu.CompilerParams(use_tc_tiling_on_sc=False),
        scratch_types=[
            pltpu.VMEM((CHUNK,), i32),
            pltpu.VMEM((CHUNK, W), f32),
            pltpu.VMEM((16, W), f32),
            pltpu.VMEM_SHARED((MP, W), f32),
        ],
    )
    def k(vals_h, idx_h, p0_h, p1_h, idxb, rows_v, zrows, acc):
        c = lax.axis_index("c")
        s = lax.axis_index("s")
        _zero2d(zrows, W)
        mpt = MP // 16

        def zb(i, _):
            pltpu.sync_copy(zrows, acc.at[pl.ds(s * mpt + i * 16, 16), :])
            return 0
        lax.fori_loop(0, mpt // 16, zb, 0)
        plsc.subcore_barrier()

        base = (c * 16 + s) * rpt

        def step(j, _):
            b = base + j * CHUNK
            pltpu.sync_copy(idx_h.at[pl.ds(b, CHUNK)], idxb)
            pltpu.sync_copy(vals_h.at[pl.ds(b, CHUNK), :], rows_v)
            pltpu.sync_copy(rows_v, acc.at[idxb], add=True)
            return 0
        lax.fori_loop(0, rpt // CHUNK, step, 0)
        plsc.subcore_barrier()

        def wout(out_h):
            pltpu.sync_copy(acc.at[pl.ds(s * mpt, mpt), :],
                            rows_v.at[pl.ds(0, mpt), :])
            pltpu.sync_copy(rows_v.at[pl.ds(0, mpt), :],
                            out_h.at[pl.ds(s * mpt, mpt), :])

        @pl.when(c == 0)
        def _():
            wout(p0_h)

        @pl.when(c == 1)
        def _():
            wout(p1_h)

    return k(vals, idx_p)


def _sc_gather(table, idx_p, W):
    rpt = NP // 32

    @functools.partial(
        pl.kernel,
        out_type=jax.ShapeDtypeStruct((NP, W), f32),
        mesh=_mesh(),
        compiler_params=pltpu.CompilerParams(use_tc_tiling_on_sc=False),
        scratch_types=[
            pltpu.VMEM((CHUNK,), i32),
            pltpu.VMEM((CHUNK, W), f32),
            pltpu.SemaphoreType.DMA,
        ],
    )
    def k(tab_h, idx_h, out_h, idxb, rows_v, sem):
        c = lax.axis_index("c")
        s = lax.axis_index("s")
        base = (c * 16 + s) * rpt

        def step(j, _):
            b = base + j * CHUNK
            pltpu.sync_copy(idx_h.at[pl.ds(b, CHUNK)], idxb)
            pltpu.async_copy(tab_h.at[idxb], rows_v, sem).wait()
            pltpu.sync_copy(rows_v, out_h.at[pl.ds(b, CHUNK), :])
            return 0
        lax.fori_loop(0, rpt // CHUNK, step, 0)

    return k(table, idx_p)


def _elu(v):
    return jnp.where(v > 0, v, jnp.exp(v) - 1.0)


def _dot(a, b):
    return jnp.dot(a, b, preferred_element_type=f32)


def _tc(fn, out_shape, *args):
    return pl.pallas_call(fn, out_shape=out_shape)(*args)


def _tc_pre(deg0, deg1, ap0_2d, ap1_2d):
    def body(d0, d1, a0, a1, dinv_r, pres_r, dp_r):
        deg = d0[...] + d1[...] + 1.0
        dinv_r[...] = lax.rsqrt(deg)
        pres = ((a0[...] + a1[...]) > 0).astype(f32)
        deg_p = jnp.sum(pres, axis=1, keepdims=True) + 1.0
        pres_r[...] = pres
        dp_r[...] = lax.rsqrt(deg_p)
    return _tc(body, (jax.ShapeDtypeStruct((NP,), f32),
                      jax.ShapeDtypeStruct((M, M), f32),
                      jax.ShapeDtypeStruct((M, 1), f32)),
               deg0, deg1, ap0_2d, ap1_2d)


def _tc0(x, Wg1, dinv_c):
    def body(x_r, w_r, dv_r, o0_r, o1_r):
        h = dv_r[...] * _dot(x_r[...], w_r[...])
        o0_r[...] = h[:, :32]
        o1_r[...] = h[:, 32:]
    return _tc(body, (jax.ShapeDtypeStruct((N, 32), f32),
                      jax.ShapeDtypeStruct((N, 32), f32)),
               x, Wg1, dinv_c)


def _tc1(s0, s1, hp0, hp1, dinv_c, bg1, fw1, fb1, Wg2):
    def body(s0_r, s1_r, h0_r, h1_r, dv_r, b_r, fw_r, fb_r, w2_r, o0_r, o1_r):
        S = jnp.concatenate([s0_r[...][:N], s1_r[...][:N]], axis=1)
        hp = jnp.concatenate([h0_r[...], h1_r[...]], axis=1)
        dv = dv_r[...]
        h1 = _elu(dv * (S + hp) + b_r[...])
        u1 = _elu(_dot(h1, fw_r[...]) + fb_r[...])
        h2p = dv * _dot(u1, w2_r[...])
        o0_r[...] = h2p[:, :128]
        o1_r[...] = h2p[:, 128:]
    return _tc(body, (jax.ShapeDtypeStruct((N, 128), f32),
                      jax.ShapeDtypeStruct((N, 128), f32)),
               s0, s1, hp0, hp1, dinv_c, bg1, fw1, fb1, Wg2)


def _tc2a(s0, s1, hp0, hp1, dinv_c, bg2, fw2, fb2):
    def body(s0_r, s1_r, h0_r, h1_r, dv_r, b_r, fw_r, fb_r, u_r):
        S = jnp.concatenate([s0_r[...][:N], s1_r[...][:N]], axis=1)
        hp = jnp.concatenate([h0_r[...], h1_r[...]], axis=1)
        h2 = _elu(dv_r[...] * (S + hp) + b_r[...])
        u_r[...] = _elu(_dot(h2, fw_r[...]) + fb_r[...])
    return _tc(body, jax.ShapeDtypeStruct((N, 256), f32),
               s0, s1, hp0, hp1, dinv_c, bg2, fw2, fb2)


def _tc2b(u2):
    def body(u_r, o_r):
        u = u_r[...]
        s1 = jnp.sum(u, axis=0, keepdims=True)
        s2 = jnp.sum(u * u, axis=0, keepdims=True)
        mu = s1 / N
        var = s2 / N - mu * mu
        y = (u - mu) * lax.rsqrt(var + 1e-5)
        aug = jnp.concatenate(
            [y, jnp.ones((N, 1), f32), jnp.zeros((N, 15), f32)], axis=1)
        o_r[...] = jnp.concatenate(
            [aug, jnp.zeros((NP - N, 272), f32)], axis=0)
    return _tc(body, jax.ShapeDtypeStruct((NP, 272), f32), u2)


def _tc3(ps0, ps1, y_aug, pres, dp, Wl1, bl1, fwl1, fbl1, Wl2, bl2,
         fwl2, fbl2, Wm1):
    def body(p0_r, p1_r, y_r, pr_r, dp_r, wl1_r, bl1_r, fw1_r, fb1_r,
             wl2_r, bl2_r, fw2_r, fb2_r, wm1_r, g_r, p_out_r, den_r):
        sums = p0_r[...][:M] + p1_r[...][:M]
        denom = jnp.maximum(sums[:, 256:257], 1.0)
        pooled = sums[:, :256] / denom
        pr = pr_r[...]
        dp = dp_r[...]

        def pconv(g, w, b):
            td = dp * _dot(g, w)
            return dp * (_dot(pr, td) + td) + b

        lx = _elu(pconv(pooled, wl1_r[...], bl1_r[...]))
        lx = _elu(_dot(lx, fw1_r[...]) + fb1_r[...])
        lx = _elu(pconv(lx, wl2_r[...], bl2_r[...]))
        lx = _elu(_dot(lx, fw2_r[...]) + fb2_r[...])
        wm1 = wm1_r[...]
        g_r[...] = _dot(lx, wm1[256:])
        p_out_r[...] = _dot(y_r[...][:N, :256], wm1[:256])
        den_r[...] = denom
    return _tc(body, (jax.ShapeDtypeStruct((M, 128), f32),
                      jax.ShapeDtypeStruct((N, 128), f32),
                      jax.ShapeDtypeStruct((M, 1), f32)),
               ps0, ps1, y_aug, pres, dp, Wl1, bl1, fwl1, fbl1,
               Wl2, bl2, fwl2, fbl2, Wm1)


def _tc4(gbc, p_mat, dinv_c):
    def body(g_r, p_r, dv_r, o0_r, o1_r):
        hp3 = dv_r[...] * (p_r[...] + g_r[...][:N])
        o0_r[...] = hp3[:, :64]
        o1_r[...] = hp3[:, 64:]
    return _tc(body, (jax.ShapeDtypeStruct((N, 64), f32),
                      jax.ShapeDtypeStruct((N, 64), f32)),
               gbc, p_mat, dinv_c)


def _tc5(s0, s1, hp0, hp1, dinv_c, bm1, fwm1, fbm1, Wm2):
    def body(s0_r, s1_r, h0_r, h1_r, dv_r, b_r, fw_r, fb_r, w2_r, o0_r, o1_r):
        S = jnp.concatenate([s0_r[...][:N], s1_r[...][:N]], axis=1)
        hp = jnp.concatenate([h0_r[...], h1_r[...]], axis=1)
        dv = dv_r[...]
        z1 = _elu(dv * (S + hp) + b_r[...])
        u = _elu(_dot(z1, fw_r[...]) + fb_r[...])
        hp4 = dv * _dot(u, w2_r[...])
        o0_r[...] = hp4[:, :16]
        o1_r[...] = hp4[:, 16:]
    return _tc(body, (jax.ShapeDtypeStruct((N, 16), f32),
                      jax.ShapeDtypeStruct((N, 16), f32)),
               s0, s1, hp0, hp1, dinv_c, bm1, fwm1, fbm1, Wm2)


def _tc6(s0, s1, hp0, hp1, dinv_c, bm2, fwm2, fbm2, W3p):
    def body(s0_r, s1_r, h0_r, h1_r, dv_r, b_r, fw_r, fb_r, w3_r, o_r):
        S = jnp.concatenate([s0_r[...][:N], s1_r[...][:N]], axis=1)
        hp = jnp.concatenate([h0_r[...], h1_r[...]], axis=1)
        dv = dv_r[...]
        h = _elu(dv * (S + hp) + b_r[...])
        u = _elu(_dot(h, fw_r[...]) + fb_r[...])
        o_r[...] = dv * _dot(u, w3_r[...])
    return _tc(body, jax.ShapeDtypeStruct((N, 16), f32),
               s0, s1, hp0, hp1, dinv_c, bm2, fwm2, fbm2, W3p)


def _tc7(s0, s1, hp5, dinv_c, b3p, fw3p, fb3p):
    def body(s0_r, s1_r, h_r, dv_r, b_r, fw_r, fb_r, o_r):
        S = s0_r[...][:N] + s1_r[...][:N]
        z = _elu(dv_r[...] * (S + h_r[...]) + b_r[...])
        z = _elu(_dot(z, fw_r[...]) + fb_r[...])
        o_r[...] = jnp.concatenate([z, jnp.zeros((NP - N, 16), f32)], axis=0)
    return _tc(body, jax.ShapeDtypeStruct((NP, 16), f32),
               s0, s1, hp5, dinv_c, b3p, fw3p, fb3p)


def _tc8(pz0, pz1, denom, pres, dp, WOp, bOp, fwOp, fbOp):
    def body(p0_r, p1_r, den_r, pr_r, dp_r, wo_r, bo_r, fw_r, fb_r, o_r):
        pooled = (p0_r[...][:M] + p1_r[...][:M]) / den_r[...]
        pr = pr_r[...]
        dp = dp_r[...]
        td = dp * _dot(pooled, wo_r[...])
        zz = _elu(dp * (_dot(pr, td) + td) + bo_r[...])
        o_r[...] = _elu(_dot(zz, fw_r[...]) + fb_r[...])
    return _tc(body, jax.ShapeDtypeStruct((M, 16), f32),
               pz0, pz1, denom, pres, dp, WOp, bOp, fwOp, fbOp)


def kernel(x, adj, num_graphs, in_batch, cluster, params):
    p = params
    row, col = adj[0], adj[1]

    W3p = jnp.zeros((32, 16), f32).at[:, :3].set(p["GCN_M3_W"])
    b3p = jnp.zeros((16,), f32).at[:3].set(p["GCN_M3_b"])
    fw3p = jnp.zeros((16, 16), f32).at[:3, :3].set(p["fc_M3_W"])
    fb3p = jnp.zeros((16,), f32).at[:3].set(p["fc_M3_b"])
    WOp = jnp.zeros((16, 16), f32).at[:3, :3].set(p["GCN_O1_W"])
    bOp = jnp.zeros((16,), f32).at[:3].set(p["GCN_O1_b"])
    fwOp = jnp.zeros((16, 16), f32).at[:3, :3].set(p["fc_O1_W"])
    fbOp = jnp.zeros((16,), f32).at[:3].set(p["fc_O1_b"])

    bc = cluster + in_batch * CN
    bc_p = jnp.concatenate([bc, jnp.zeros(NP - N, i32)])
    deg0, deg1, ap0, ap1 = _sc_prep(row, col, bc)
    ap0_2d = ap0[: M * M].reshape(M, M)
    ap1_2d = ap1[: M * M].reshape(M, M)
    dinv1d, pres, dp = _tc_pre(deg0, deg1, ap0_2d, ap1_2d)
    dinv_c = dinv1d[:N].reshape(N, 1)

    hp1a, hp1b = _tc0(x, p["GCN_G1_W"], dinv_c)
    s1a, s1b = _sc_agg2(32, hp1a, hp1b, row, col)
    hp2a, hp2b = _tc1(s1a, s1b, hp1a, hp1b, dinv_c, p["GCN_G1_b"],
                      p["fc_G1_W"], p["fc_G1_b"], p["GCN_G2_W"])
    s2a, s2b = _sc_agg2(128, hp2a, hp2b, row, col)
    u2 = _tc2a(s2a, s2b, hp2a, hp2b, dinv_c, p["GCN_G2_b"],
               p["fc_G2_W"], p["fc_G2_b"])
    y_aug = _tc2b(u2)

    ps0, ps1 = _sc_pool(y_aug, bc_p, 272)
    G, P, denom = _tc3(ps0, ps1, y_aug, pres, dp,
                       p["GCN_L1_W"], p["GCN_L1_b"], p["fc_L1_W"], p["fc_L1_b"],
                       p["GCN_L2_W"], p["GCN_L2_b"], p["fc_L2_W"], p["fc_L2_b"],
                       p["GCN_M1_W"])
    Gbc = _sc_gather(G, bc_p, 128)
    hp3a, hp3b = _tc4(Gbc, P, dinv_c)
    s3a, s3b = _sc_agg2(64, hp3a, hp3b, row, col)
    hp4a, hp4b = _tc5(s3a, s3b, hp3a, hp3b, dinv_c, p["GCN_M1_b"],
                      p["fc_M1_W"], p["fc_M1_b"], p["GCN_M2_W"])
    s4a, s4b = _sc_agg2(16, hp4a, hp4b, row, col)
    hp5 = _tc6(s4a, s4b, hp4a, hp4b, dinv_c, p["GCN_M2_b"],
               p["fc_M2_W"], p["fc_M2_b"], W3p)
    s5a, s5b = _sc_agg_eb(16, hp5, row, col)
    z8 = _tc7(s5a, s5b, hp5, dinv_c, b3p, fw3p, fb3p)

    pz0, pz1 = _sc_pool(z8, bc_p, 16)
    zz8 = _tc8(pz0, pz1, denom, pres, dp, WOp, bOp, fwOp, fbOp)
    orows = _sc_gather(zz8, bc_p, 16)
    return (orows[:N, :3], zz8[:, :3])


# speedup vs baseline: 25.1031x; 1.7514x over previous
import functools

import jax
import jax.numpy as jnp
from jax import lax
from jax.experimental import pallas as pl
from jax.experimental.pallas import tpu as pltpu
from jax.experimental.pallas import tpu_sc as plsc

N = 10000
NP = 10240
E = 320000
M = 800
MP = 1024
CN = 100
DUMP = M * M
APLEN = 640256
CHUNK = 80

f32 = jnp.float32
i32 = jnp.int32


def _mesh():
    return plsc.VectorSubcoreMesh(core_axis_name="c", subcore_axis_name="s")


def _fill1d(ref, n, val, dtype):
    def body(i, _):
        ref[pl.ds(i * 16, 16)] = jnp.full((16,), val, dtype)
        return 0
    lax.fori_loop(0, n // 16, body, 0)


def _zero2d(ref, w):
    def body(i, _):
        def inner(j, c):
            ref[i, pl.ds(j * 16, 16)] = jnp.zeros((16,), f32)
            return c
        lax.fori_loop(0, w // 16, inner, 0)
        return 0
    lax.fori_loop(0, 16, body, 0)


def _sc_prep(row2, col2, bc_p):
    it = E // 32 // CHUNK

    @functools.partial(
        pl.kernel,
        out_type=(
            jax.ShapeDtypeStruct((NP,), f32),
            jax.ShapeDtypeStruct((NP,), f32),
            jax.ShapeDtypeStruct((APLEN,), f32),
            jax.ShapeDtypeStruct((APLEN,), f32),
        ),
        mesh=_mesh(),
        compiler_params=pltpu.CompilerParams(use_tc_tiling_on_sc=False),
        scratch_types=[
            pltpu.VMEM((it, CHUNK), i32),
            pltpu.VMEM((it, CHUNK), i32),
            pltpu.VMEM((CHUNK,), i32),
            pltpu.VMEM((CHUNK,), i32),
            pltpu.VMEM((CHUNK,), i32),
            pltpu.VMEM((CHUNK,), i32),
            pltpu.VMEM((CHUNK,), i32),
            pltpu.VMEM((CHUNK,), i32),
            pltpu.VMEM((CHUNK,), f32),
            pltpu.VMEM((4096,), f32),
            pltpu.VMEM_SHARED((NP,), f32),
            pltpu.VMEM_SHARED((APLEN,), f32),
            pltpu.SemaphoreType.DMA,
            pltpu.SemaphoreType.DMA,
            pltpu.SemaphoreType.DMA,
            pltpu.SemaphoreType.DMA,
        ],
    )
    def k(row_h, col_h, bc_h, deg0_h, deg1_h, ap0_h, ap1_h,
          rowix, colix, e0a, e1a, e0b, e1b, apv0, apv1, ones_v, zline,
          acc_deg, acc_ap, semA, semB, semGA, semGB):
        c = lax.axis_index("c")
        s = lax.axis_index("s")
        _fill1d(ones_v, CHUNK, 1.0, f32)
        _fill1d(zline, 4096, 0.0, f32)
        tbase = (c * 16 + s) * it
        pltpu.sync_copy(row_h.at[pl.ds(tbase, it), :], rowix)
        pltpu.sync_copy(col_h.at[pl.ds(tbase, it), :], colix)

        pltpu.sync_copy(zline.at[pl.ds(0, NP // 16)],
                        acc_deg.at[pl.ds(s * (NP // 16), NP // 16)])
        span = APLEN // 16
        def zap(j, _):
            pltpu.sync_copy(zline, acc_ap.at[pl.ds(s * span + j * 4096, 4096)])
            return 0
        lax.fori_loop(0, 9, zap, 0)
        pltpu.sync_copy(zline.at[pl.ds(0, 3152)],
                        acc_ap.at[pl.ds(s * span + 9 * 4096, 3152)])
        plsc.subcore_barrier()

        def fire_g(j, e0, e1, semG):
            pltpu.async_copy(bc_h.at[rowix.at[j]], e0, semG)
            pltpu.async_copy(bc_h.at[colix.at[j]], e1, semG)

        def waitg(j, e0, e1, semG):
            pltpu.make_async_copy(bc_h.at[rowix.at[j]], e0, semG).wait()
            pltpu.make_async_copy(bc_h.at[colix.at[j]], e1, semG).wait()

        def mkpid(e0v, e1v, apv):
            def grp(g, _):
                e0 = e0v[pl.ds(g * 16, 16)]
                e1 = e1v[pl.ds(g * 16, 16)]
                pid = jnp.where(e0 != e1, e1 * M + e0, DUMP)
                apv[pl.ds(g * 16, 16)] = pid
                return 0
            lax.fori_loop(0, CHUNK // 16, grp, 0)

        fire_g(0, e0a, e1a, semGA)

        def pair(j2, _):
            jA = j2 * 2

            @pl.when(jA + 1 < it)
            def _():
                fire_g(jA + 1, e0b, e1b, semGB)
            waitg(jA, e0a, e1a, semGA)
            mkpid(e0a, e1a, apv0)
            gA = pltpu.async_copy(ones_v, acc_ap.at[apv0], semA, add=True)
            pltpu.sync_copy(ones_v, acc_deg.at[colix.at[jA]], add=True)

            @pl.when(jA + 2 < it)
            def _():
                fire_g(jA + 2, e0a, e1a, semGA)
            gA.wait()

            @pl.when(jA + 1 < it)
            def _():
                waitg(jA + 1, e0b, e1b, semGB)
                mkpid(e0b, e1b, apv1)
                gB = pltpu.async_copy(ones_v, acc_ap.at[apv1], semB, add=True)
                pltpu.sync_copy(ones_v, acc_deg.at[colix.at[jA + 1]], add=True)
                gB.wait()
            return 0
        lax.fori_loop(0, (it + 1) // 2, pair, 0)
        plsc.subcore_barrier()

        rw = NP // 16

        def wout(deg_h, ap_h):
            pltpu.sync_copy(acc_deg.at[pl.ds(s * rw, rw)],
                            zline.at[pl.ds(0, rw)])
            pltpu.sync_copy(zline.at[pl.ds(0, rw)],
                            deg_h.at[pl.ds(s * rw, rw)])

            def wr(j, _):
                pltpu.sync_copy(acc_ap.at[pl.ds(s * span + j * 4096, 4096)],
                                zline)
                pltpu.sync_copy(zline,
                                ap_h.at[pl.ds(s * span + j * 4096, 4096)])
                return 0
            lax.fori_loop(0, 9, wr, 0)
            pltpu.sync_copy(acc_ap.at[pl.ds(s * span + 9 * 4096, 3152)],
                            zline.at[pl.ds(0, 3152)])
            pltpu.sync_copy(zline.at[pl.ds(0, 3152)],
                            ap_h.at[pl.ds(s * span + 9 * 4096, 3152)])

        @pl.when(c == 0)
        def _():
            wout(deg0_h, ap0_h)

        @pl.when(c == 1)
        def _():
            wout(deg1_h, ap1_h)

    return k(row2, col2, bc_p)


def _edge_pipeline(hp_h, acc, rowix, colix, rows0, rows1, sem0, sem1, it):
    pltpu.async_copy(hp_h.at[rowix.at[0]], rows0, sem0)

    def pair(j2, _):
        jA = j2 * 2

        @pl.when(jA + 1 < it)
        def _():
            pltpu.async_copy(hp_h.at[rowix.at[jA + 1]], rows1, sem1)

        pltpu.make_async_copy(hp_h.at[rowix.at[jA]], rows0, sem0).wait()
        pltpu.sync_copy(rows0, acc.at[colix.at[jA]], add=True)

        @pl.when(jA + 2 < it)
        def _():
            pltpu.async_copy(hp_h.at[rowix.at[jA + 2]], rows0, sem0)

        @pl.when(jA + 1 < it)
        def _():
            pltpu.make_async_copy(hp_h.at[rowix.at[jA + 1]], rows1, sem1).wait()
            pltpu.sync_copy(rows1, acc.at[colix.at[jA + 1]], add=True)
        return 0
    lax.fori_loop(0, (it + 1) // 2, pair, 0)


def _sc_agg2(W, hp0, hp1, row2, col2):
    it = E // 16 // CHUNK
    BCH = 25
    NBLK = it // BCH

    @functools.partial(
        pl.kernel,
        out_type=(
            jax.ShapeDtypeStruct((NP, W), f32),
            jax.ShapeDtypeStruct((NP, W), f32),
        ),
        mesh=_mesh(),
        compiler_params=pltpu.CompilerParams(use_tc_tiling_on_sc=False),
        scratch_types=[
            pltpu.VMEM((BCH, CHUNK), i32),
            pltpu.VMEM((BCH, CHUNK), i32),
            pltpu.VMEM((BCH, CHUNK), i32),
            pltpu.VMEM((BCH, CHUNK), i32),
            pltpu.VMEM((CHUNK, W), f32),
            pltpu.VMEM((CHUNK, W), f32),
            pltpu.VMEM((16, W), f32),
            pltpu.VMEM_SHARED((NP, W), f32),
            pltpu.SemaphoreType.DMA,
            pltpu.SemaphoreType.DMA,
            pltpu.SemaphoreType.DMA,
            pltpu.SemaphoreType.DMA,
        ],
    )
    def k(row_h, col_h, hp0_h, hp1_h, s0_h, s1_h,
          rixP, cixP, rixQ, cixQ, rows0, rows1, zrows, acc,
          sem0, sem1, semIP, semIQ):
        c = lax.axis_index("c")
        s = lax.axis_index("s")
        _zero2d(zrows, W)
        rpt = NP // 16

        def zb(i, _):
            pltpu.sync_copy(zrows, acc.at[pl.ds(s * rpt + i * 16, 16), :])
            return 0
        lax.fori_loop(0, rpt // 16, zb, 0)
        plsc.subcore_barrier()

        def ldblk(b, rix, cix):
            pltpu.sync_copy(row_h.at[pl.ds(s * it + b * BCH, BCH), :], rix)
            pltpu.sync_copy(col_h.at[pl.ds(s * it + b * BCH, BCH), :], cix)

        def fireblk(b, rix, cix, semI):
            pltpu.async_copy(row_h.at[pl.ds(s * it + b * BCH, BCH), :], rix, semI)
            pltpu.async_copy(col_h.at[pl.ds(s * it + b * BCH, BCH), :], cix, semI)

        def waitblk(b, rix, cix, semI):
            pltpu.make_async_copy(row_h.at[pl.ds(s * it + b * BCH, BCH), :], rix, semI).wait()
            pltpu.make_async_copy(col_h.at[pl.ds(s * it + b * BCH, BCH), :], cix, semI).wait()

        def work(hp_h):
            ldblk(0, rixP, cixP)
            fireblk(1, rixQ, cixQ, semIQ)

            def bpair(k2, _):
                b = k2 * 2
                _edge_pipeline(hp_h, acc, rixP, cixP, rows0, rows1,
                               sem0, sem1, BCH)
                waitblk(b + 1, rixQ, cixQ, semIQ)

                @pl.when(b + 2 < NBLK)
                def _():
                    fireblk(b + 2, rixP, cixP, semIP)
                _edge_pipeline(hp_h, acc, rixQ, cixQ, rows0, rows1,
                               sem0, sem1, BCH)

                @pl.when(b + 2 < NBLK)
                def _():
                    waitblk(b + 2, rixP, cixP, semIP)

                    @pl.when(b + 3 < NBLK)
                    def _():
                        fireblk(b + 3, rixQ, cixQ, semIQ)
                return 0
            lax.fori_loop(0, NBLK // 2, bpair, 0)

        @pl.when(c == 0)
        def _():
            work(hp0_h)

        @pl.when(c == 1)
        def _():
            work(hp1_h)
        plsc.subcore_barrier()

        def wout(out_h):
            def w(i, _):
                r0 = s * rpt + i * CHUNK
                pltpu.sync_copy(acc.at[pl.ds(r0, CHUNK), :], rows0)
                pltpu.sync_copy(rows0, out_h.at[pl.ds(r0, CHUNK), :])
                return 0
            lax.fori_loop(0, rpt // CHUNK, w, 0)

        @pl.when(c == 0)
        def _():
            wout(s0_h)

        @pl.when(c == 1)
        def _():
            wout(s1_h)

    return k(row2, col2, hp0, hp1)


def _sc_agg_eb(W, hp, row2, col2):
    it = E // 32 // CHUNK

    @functools.partial(
        pl.kernel,
        out_type=(
            jax.ShapeDtypeStruct((NP, W), f32),
            jax.ShapeDtypeStruct((NP, W), f32),
        ),
        mesh=_mesh(),
        compiler_params=pltpu.CompilerParams(use_tc_tiling_on_sc=False),
        scratch_types=[
            pltpu.VMEM((E // 32 // CHUNK, CHUNK), i32),
            pltpu.VMEM((E // 32 // CHUNK, CHUNK), i32),
            pltpu.VMEM((CHUNK, W), f32),
            pltpu.VMEM((CHUNK, W), f32),
            pltpu.VMEM((16, W), f32),
            pltpu.VMEM_SHARED((NP, W), f32),
            pltpu.SemaphoreType.DMA,
            pltpu.SemaphoreType.DMA,
        ],
    )
    def k(row_h, col_h, hp_h, s0_h, s1_h,
          rowix, colix, rows0, rows1, zrows, acc, sem0, sem1):
        c = lax.axis_index("c")
        s = lax.axis_index("s")
        tbase = (c * 16 + s) * it
        pltpu.sync_copy(row_h.at[pl.ds(tbase, it), :], rowix)
        pltpu.sync_copy(col_h.at[pl.ds(tbase, it), :], colix)
        _zero2d(zrows, W)
        rpt = NP // 16

        def zb(i, _):
            pltpu.sync_copy(zrows, acc.at[pl.ds(s * rpt + i * 16, 16), :])
            return 0
        lax.fori_loop(0, rpt // 16, zb, 0)
        plsc.subcore_barrier()

        _edge_pipeline(hp_h, acc, rowix, colix, rows0, rows1,
                       sem0, sem1, it)
        plsc.subcore_barrier()

        def wout(out_h):
            def w(i, _):
                r0 = s * rpt + i * CHUNK
                pltpu.sync_copy(acc.at[pl.ds(r0, CHUNK), :], rows0)
                pltpu.sync_copy(rows0, out_h.at[pl.ds(r0, CHUNK), :])
                return 0
            lax.fori_loop(0, rpt // CHUNK, w, 0)

        @pl.when(c == 0)
        def _():
            wout(s0_h)

        @pl.when(c == 1)
        def _():
            wout(s1_h)

    return k(row2, col2, hp)


def _sc_pool(vals, idx_p, W):
    rpt = NP // 32

    @functools.partial(
        pl.kernel,
        out_type=(
            jax.ShapeDtypeStruct((MP, W), f32),
            jax.ShapeDtypeStruct((MP, W), f32),
        ),
        mesh=_mesh(),
        compiler_params=pltpu.CompilerParams(use_tc_tiling_on_sc=False),
        scratch_types=[
            pltpu.VMEM((CHUNK,), i32),
            pltpu.VMEM((CHUNK, W), f32),
            pltpu.VMEM((16, W), f32),
            pltpu.VMEM_SHARED((MP, W), f32),
        ],
    )
    def k(vals_h, idx_h, p0_h, p1_h, idxb, rows_v, zrows, acc):
        c = lax.axis_index("c")
        s = lax.axis_index("s")
        _zero2d(zrows, W)
        mpt = MP // 16

        def zb(i, _):
            pltpu.sync_copy(zrows, acc.at[pl.ds(s * mpt + i * 16, 16), :])
            return 0
        lax.fori_loop(0, mpt // 16, zb, 0)
        plsc.subcore_barrier()

        base = (c * 16 + s) * rpt

        def step(j, _):
            b = base + j * CHUNK
            pltpu.sync_copy(idx_h.at[pl.ds(b, CHUNK)], idxb)
            pltpu.sync_copy(vals_h.at[pl.ds(b, CHUNK), :], rows_v)
            pltpu.sync_copy(rows_v, acc.at[idxb], add=True)
            return 0
        lax.fori_loop(0, rpt // CHUNK, step, 0)
        plsc.subcore_barrier()

        def wout(out_h):
            pltpu.sync_copy(acc.at[pl.ds(s * mpt, mpt), :],
                            rows_v.at[pl.ds(0, mpt), :])
            pltpu.sync_copy(rows_v.at[pl.ds(0, mpt), :],
                            out_h.at[pl.ds(s * mpt, mpt), :])

        @pl.when(c == 0)
        def _():
            wout(p0_h)

        @pl.when(c == 1)
        def _():
            wout(p1_h)

    return k(vals, idx_p)


def _sc_gather(table, idx_p, W):
    rpt = NP // 32

    @functools.partial(
        pl.kernel,
        out_type=jax.ShapeDtypeStruct((NP, W), f32),
        mesh=_mesh(),
        compiler_params=pltpu.CompilerParams(use_tc_tiling_on_sc=False),
        scratch_types=[
            pltpu.VMEM((CHUNK,), i32),
            pltpu.VMEM((CHUNK, W), f32),
            pltpu.SemaphoreType.DMA,
        ],
    )
    def k(tab_h, idx_h, out_h, idxb, rows_v, sem):
        c = lax.axis_index("c")
        s = lax.axis_index("s")
        base = (c * 16 + s) * rpt

        def step(j, _):
            b = base + j * CHUNK
            pltpu.sync_copy(idx_h.at[pl.ds(b, CHUNK)], idxb)
            pltpu.async_copy(tab_h.at[idxb], rows_v, sem).wait()
            pltpu.sync_copy(rows_v, out_h.at[pl.ds(b, CHUNK), :])
            return 0
        lax.fori_loop(0, rpt // CHUNK, step, 0)

    return k(table, idx_p)


def _elu(v):
    return jnp.where(v > 0, v, jnp.exp(v) - 1.0)


def _dot(a, b):
    return jnp.dot(a, b, preferred_element_type=f32)


def _tc(fn, out_shape, *args):
    return pl.pallas_call(fn, out_shape=out_shape)(*args)


def _tc_pre(deg0, deg1, ap0_2d, ap1_2d):
    def body(d0, d1, a0, a1, dinv_r, pres_r, dp_r):
        deg = d0[...] + d1[...] + 1.0
        dinv_r[...] = lax.rsqrt(deg)
        pres = ((a0[...] + a1[...]) > 0).astype(f32)
        deg_p = jnp.sum(pres, axis=1, keepdims=True) + 1.0
        pres_r[...] = pres
        dp_r[...] = lax.rsqrt(deg_p)
    return _tc(body, (jax.ShapeDtypeStruct((NP,), f32),
                      jax.ShapeDtypeStruct((M, M), f32),
                      jax.ShapeDtypeStruct((M, 1), f32)),
               deg0, deg1, ap0_2d, ap1_2d)


def _tc0(x, Wg1, dinv_c):
    def body(x_r, w_r, dv_r, o0_r, o1_r):
        h = dv_r[...] * _dot(x_r[...], w_r[...])
        o0_r[...] = h[:, :32]
        o1_r[...] = h[:, 32:]
    return _tc(body, (jax.ShapeDtypeStruct((N, 32), f32),
                      jax.ShapeDtypeStruct((N, 32), f32)),
               x, Wg1, dinv_c)


def _tc1(s0, s1, hp0, hp1, dinv_c, bg1, fw1, fb1, Wg2):
    def body(s0_r, s1_r, h0_r, h1_r, dv_r, b_r, fw_r, fb_r, w2_r, o0_r, o1_r):
        S = jnp.concatenate([s0_r[...][:N], s1_r[...][:N]], axis=1)
        hp = jnp.concatenate([h0_r[...], h1_r[...]], axis=1)
        dv = dv_r[...]
        h1 = _elu(dv * (S + hp) + b_r[...])
        u1 = _elu(_dot(h1, fw_r[...]) + fb_r[...])
        h2p = dv * _dot(u1, w2_r[...])
        o0_r[...] = h2p[:, :128]
        o1_r[...] = h2p[:, 128:]
    return _tc(body, (jax.ShapeDtypeStruct((N, 128), f32),
                      jax.ShapeDtypeStruct((N, 128), f32)),
               s0, s1, hp0, hp1, dinv_c, bg1, fw1, fb1, Wg2)


def _tc2a(s0, s1, hp0, hp1, dinv_c, bg2, fw2, fb2):
    def body(s0_r, s1_r, h0_r, h1_r, dv_r, b_r, fw_r, fb_r, u_r):
        S = jnp.concatenate([s0_r[...][:N], s1_r[...][:N]], axis=1)
        hp = jnp.concatenate([h0_r[...], h1_r[...]], axis=1)
        h2 = _elu(dv_r[...] * (S + hp) + b_r[...])
        u_r[...] = _elu(_dot(h2, fw_r[...]) + fb_r[...])
    return _tc(body, jax.ShapeDtypeStruct((N, 256), f32),
               s0, s1, hp0, hp1, dinv_c, bg2, fw2, fb2)


def _tc2b(u2):
    def body(u_r, o_r):
        u = u_r[...]
        s1 = jnp.sum(u, axis=0, keepdims=True)
        s2 = jnp.sum(u * u, axis=0, keepdims=True)
        mu = s1 / N
        var = s2 / N - mu * mu
        y = (u - mu) * lax.rsqrt(var + 1e-5)
        aug = jnp.concatenate(
            [y, jnp.ones((N, 1), f32), jnp.zeros((N, 15), f32)], axis=1)
        o_r[...] = jnp.concatenate(
            [aug, jnp.zeros((NP - N, 272), f32)], axis=0)
    return _tc(body, jax.ShapeDtypeStruct((NP, 272), f32), u2)


def _tc3(ps0, ps1, y_aug, pres, dp, Wl1, bl1, fwl1, fbl1, Wl2, bl2,
         fwl2, fbl2, Wm1):
    def body(p0_r, p1_r, y_r, pr_r, dp_r, wl1_r, bl1_r, fw1_r, fb1_r,
             wl2_r, bl2_r, fw2_r, fb2_r, wm1_r, g_r, p_out_r, den_r):
        sums = p0_r[...][:M] + p1_r[...][:M]
        denom = jnp.maximum(sums[:, 256:257], 1.0)
        pooled = sums[:, :256] / denom
        pr = pr_r[...]
        dp = dp_r[...]

        def pconv(g, w, b):
            td = dp * _dot(g, w)
            return dp * (_dot(pr, td) + td) + b

        lx = _elu(pconv(pooled, wl1_r[...], bl1_r[...]))
        lx = _elu(_dot(lx, fw1_r[...]) + fb1_r[...])
        lx = _elu(pconv(lx, wl2_r[...], bl2_r[...]))
        lx = _elu(_dot(lx, fw2_r[...]) + fb2_r[...])
        wm1 = wm1_r[...]
        g_r[...] = _dot(lx, wm1[256:])
        p_out_r[...] = _dot(y_r[...][:N, :256], wm1[:256])
        den_r[...] = denom
    return _tc(body, (jax.ShapeDtypeStruct((M, 128), f32),
                      jax.ShapeDtypeStruct((N, 128), f32),
                      jax.ShapeDtypeStruct((M, 1), f32)),
               ps0, ps1, y_aug, pres, dp, Wl1, bl1, fwl1, fbl1,
               Wl2, bl2, fwl2, fbl2, Wm1)


def _tc4(gbc, p_mat, dinv_c):
    def body(g_r, p_r, dv_r, o0_r, o1_r):
        hp3 = dv_r[...] * (p_r[...] + g_r[...][:N])
        o0_r[...] = hp3[:, :64]
        o1_r[...] = hp3[:, 64:]
    return _tc(body, (jax.ShapeDtypeStruct((N, 64), f32),
                      jax.ShapeDtypeStruct((N, 64), f32)),
               gbc, p_mat, dinv_c)


def _tc5(s0, s1, hp0, hp1, dinv_c, bm1, fwm1, fbm1, Wm2):
    def body(s0_r, s1_r, h0_r, h1_r, dv_r, b_r, fw_r, fb_r, w2_r, o0_r, o1_r):
        S = jnp.concatenate([s0_r[...][:N], s1_r[...][:N]], axis=1)
        hp = jnp.concatenate([h0_r[...], h1_r[...]], axis=1)
        dv = dv_r[...]
        z1 = _elu(dv * (S + hp) + b_r[...])
        u = _elu(_dot(z1, fw_r[...]) + fb_r[...])
        hp4 = dv * _dot(u, w2_r[...])
        o0_r[...] = hp4[:, :16]
        o1_r[...] = hp4[:, 16:]
    return _tc(body, (jax.ShapeDtypeStruct((N, 16), f32),
                      jax.ShapeDtypeStruct((N, 16), f32)),
               s0, s1, hp0, hp1, dinv_c, bm1, fwm1, fbm1, Wm2)


def _tc6(s0, s1, hp0, hp1, dinv_c, bm2, fwm2, fbm2, W3p):
    def body(s0_r, s1_r, h0_r, h1_r, dv_r, b_r, fw_r, fb_r, w3_r, o_r):
        S = jnp.concatenate([s0_r[...][:N], s1_r[...][:N]], axis=1)
        hp = jnp.concatenate([h0_r[...], h1_r[...]], axis=1)
        dv = dv_r[...]
        h = _elu(dv * (S + hp) + b_r[...])
        u = _elu(_dot(h, fw_r[...]) + fb_r[...])
        o_r[...] = dv * _dot(u, w3_r[...])
    return _tc(body, jax.ShapeDtypeStruct((N, 16), f32),
               s0, s1, hp0, hp1, dinv_c, bm2, fwm2, fbm2, W3p)


def _tc7(s0, s1, hp5, dinv_c, b3p, fw3p, fb3p):
    def body(s0_r, s1_r, h_r, dv_r, b_r, fw_r, fb_r, o_r):
        S = s0_r[...][:N] + s1_r[...][:N]
        z = _elu(dv_r[...] * (S + h_r[...]) + b_r[...])
        z = _elu(_dot(z, fw_r[...]) + fb_r[...])
        o_r[...] = jnp.concatenate([z, jnp.zeros((NP - N, 16), f32)], axis=0)
    return _tc(body, jax.ShapeDtypeStruct((NP, 16), f32),
               s0, s1, hp5, dinv_c, b3p, fw3p, fb3p)


def _tc8(pz0, pz1, denom, pres, dp, WOp, bOp, fwOp, fbOp):
    def body(p0_r, p1_r, den_r, pr_r, dp_r, wo_r, bo_r, fw_r, fb_r, o_r):
        pooled = (p0_r[...][:M] + p1_r[...][:M]) / den_r[...]
        pr = pr_r[...]
        dp = dp_r[...]
        td = dp * _dot(pooled, wo_r[...])
        zz = _elu(dp * (_dot(pr, td) + td) + bo_r[...])
        o_r[...] = _elu(_dot(zz, fw_r[...]) + fb_r[...])
    return _tc(body, jax.ShapeDtypeStruct((M, 16), f32),
               pz0, pz1, denom, pres, dp, WOp, bOp, fwOp, fbOp)


def kernel(x, adj, num_graphs, in_batch, cluster, params):
    p = params
    row, col = adj[0], adj[1]

    W3p = jnp.zeros((32, 16), f32).at[:, :3].set(p["GCN_M3_W"])
    b3p = jnp.zeros((16,), f32).at[:3].set(p["GCN_M3_b"])
    fw3p = jnp.zeros((16, 16), f32).at[:3, :3].set(p["fc_M3_W"])
    fb3p = jnp.zeros((16,), f32).at[:3].set(p["fc_M3_b"])
    WOp = jnp.zeros((16, 16), f32).at[:3, :3].set(p["GCN_O1_W"])
    bOp = jnp.zeros((16,), f32).at[:3].set(p["GCN_O1_b"])
    fwOp = jnp.zeros((16, 16), f32).at[:3, :3].set(p["fc_O1_W"])
    fbOp = jnp.zeros((16,), f32).at[:3].set(p["fc_O1_b"])

    bc = cluster + in_batch * CN
    bc_p = jnp.concatenate([bc, jnp.zeros(NP - N, i32)])
    row2 = row.reshape(E // CHUNK, CHUNK)
    col2 = col.reshape(E // CHUNK, CHUNK)
    deg0, deg1, ap0, ap1 = _sc_prep(row2, col2, bc_p)
    ap0_2d = ap0[: M * M].reshape(M, M)
    ap1_2d = ap1[: M * M].reshape(M, M)
    dinv1d, pres, dp = _tc_pre(deg0, deg1, ap0_2d, ap1_2d)
    dinv_c = dinv1d[:N].reshape(N, 1)

    hp1a, hp1b = _tc0(x, p["GCN_G1_W"], dinv_c)
    s1a, s1b = _sc_agg2(32, hp1a, hp1b, row2, col2)
    hp2a, hp2b = _tc1(s1a, s1b, hp1a, hp1b, dinv_c, p["GCN_G1_b"],
                      p["fc_G1_W"], p["fc_G1_b"], p["GCN_G2_W"])
    s2a, s2b = _sc_agg2(128, hp2a, hp2b, row2, col2)
    u2 = _tc2a(s2a, s2b, hp2a, hp2b, dinv_c, p["GCN_G2_b"],
               p["fc_G2_W"], p["fc_G2_b"])
    y_aug = _tc2b(u2)

    ps0, ps1 = _sc_pool(y_aug, bc_p, 272)
    G, P, denom = _tc3(ps0, ps1, y_aug, pres, dp,
                       p["GCN_L1_W"], p["GCN_L1_b"], p["fc_L1_W"], p["fc_L1_b"],
                       p["GCN_L2_W"], p["GCN_L2_b"], p["fc_L2_W"], p["fc_L2_b"],
                       p["GCN_M1_W"])
    Gbc = _sc_gather(G, bc_p, 128)
    hp3a, hp3b = _tc4(Gbc, P, dinv_c)
    s3a, s3b = _sc_agg2(64, hp3a, hp3b, row2, col2)
    hp4a, hp4b = _tc5(s3a, s3b, hp3a, hp3b, dinv_c, p["GCN_M1_b"],
                      p["fc_M1_W"], p["fc_M1_b"], p["GCN_M2_W"])
    s4a, s4b = _sc_agg2(16, hp4a, hp4b, row2, col2)
    hp5 = _tc6(s4a, s4b, hp4a, hp4b, dinv_c, p["GCN_M2_b"],
               p["fc_M2_W"], p["fc_M2_b"], W3p)
    s5a, s5b = _sc_agg_eb(16, hp5, row2, col2)
    z8 = _tc7(s5a, s5b, hp5, dinv_c, b3p, fw3p, fb3p)

    pz0, pz1 = _sc_pool(z8, bc_p, 16)
    zz8 = _tc8(pz0, pz1, denom, pres, dp, WOp, bOp, fwOp, fbOp)
    orows = _sc_gather(zz8, bc_p, 16)
    return (orows[:N, :3], zz8[:, :3])


# speedup vs baseline: 28.7178x; 1.1440x over previous
import functools

import jax
import jax.numpy as jnp
from jax import lax
from jax.experimental import pallas as pl
from jax.experimental.pallas import tpu as pltpu
from jax.experimental.pallas import tpu_sc as plsc

N = 10000
NP = 10240
E = 320000
M = 800
MP = 1024
CN = 100
DUMP = M * M
APLEN = 640256
CHUNK = 80

f32 = jnp.float32
i32 = jnp.int32


def _mesh():
    return plsc.VectorSubcoreMesh(core_axis_name="c", subcore_axis_name="s")


def _fill1d(ref, n, val, dtype):
    def body(i, _):
        ref[pl.ds(i * 16, 16)] = jnp.full((16,), val, dtype)
        return 0
    lax.fori_loop(0, n // 16, body, 0)


def _zero2d(ref, nrows, w):
    def body(i, _):
        def inner(j, c):
            ref[i, pl.ds(j * 16, 16)] = jnp.zeros((16,), f32)
            return c
        lax.fori_loop(0, w // 16, inner, 0)
        return 0
    lax.fori_loop(0, nrows, body, 0)


def _sc_prep(row2, col2, bc_p):
    it = E // 32 // CHUNK

    @functools.partial(
        pl.kernel,
        out_type=(
            jax.ShapeDtypeStruct((NP,), f32),
            jax.ShapeDtypeStruct((NP,), f32),
            jax.ShapeDtypeStruct((APLEN,), f32),
            jax.ShapeDtypeStruct((APLEN,), f32),
        ),
        mesh=_mesh(),
        compiler_params=pltpu.CompilerParams(use_tc_tiling_on_sc=False),
        scratch_types=[
            pltpu.VMEM((it, CHUNK), i32),
            pltpu.VMEM((it, CHUNK), i32),
            pltpu.VMEM((CHUNK,), i32),
            pltpu.VMEM((CHUNK,), i32),
            pltpu.VMEM((CHUNK,), i32),
            pltpu.VMEM((CHUNK,), i32),
            pltpu.VMEM((CHUNK,), i32),
            pltpu.VMEM((CHUNK,), i32),
            pltpu.VMEM((CHUNK,), f32),
            pltpu.VMEM((4096,), f32),
            pltpu.VMEM_SHARED((NP,), f32),
            pltpu.VMEM_SHARED((APLEN,), f32),
            pltpu.SemaphoreType.DMA,
            pltpu.SemaphoreType.DMA,
            pltpu.SemaphoreType.DMA,
            pltpu.SemaphoreType.DMA,
        ],
    )
    def k(row_h, col_h, bc_h, deg0_h, deg1_h, ap0_h, ap1_h,
          rowix, colix, e0a, e1a, e0b, e1b, apv0, apv1, ones_v, zline,
          acc_deg, acc_ap, semA, semB, semGA, semGB):
        c = lax.axis_index("c")
        s = lax.axis_index("s")
        _fill1d(ones_v, CHUNK, 1.0, f32)
        _fill1d(zline, 4096, 0.0, f32)
        tbase = (c * 16 + s) * it
        pltpu.sync_copy(row_h.at[pl.ds(tbase, it), :], rowix)
        pltpu.sync_copy(col_h.at[pl.ds(tbase, it), :], colix)

        pltpu.sync_copy(zline.at[pl.ds(0, NP // 16)],
                        acc_deg.at[pl.ds(s * (NP // 16), NP // 16)])
        span = APLEN // 16
        def zap(j, _):
            pltpu.sync_copy(zline, acc_ap.at[pl.ds(s * span + j * 4096, 4096)])
            return 0
        lax.fori_loop(0, 9, zap, 0)
        pltpu.sync_copy(zline.at[pl.ds(0, 3152)],
                        acc_ap.at[pl.ds(s * span + 9 * 4096, 3152)])
        plsc.subcore_barrier()

        def fire_g(j, e0, e1, semG):
            pltpu.async_copy(bc_h.at[rowix.at[j]], e0, semG)
            pltpu.async_copy(bc_h.at[colix.at[j]], e1, semG)

        def waitg(j, e0, e1, semG):
            pltpu.make_async_copy(bc_h.at[rowix.at[j]], e0, semG).wait()
            pltpu.make_async_copy(bc_h.at[colix.at[j]], e1, semG).wait()

        def mkpid(e0v, e1v, apv):
            def grp(g, _):
                e0 = e0v[pl.ds(g * 16, 16)]
                e1 = e1v[pl.ds(g * 16, 16)]
                pid = jnp.where(e0 != e1, e1 * M + e0, DUMP)
                apv[pl.ds(g * 16, 16)] = pid
                return 0
            lax.fori_loop(0, CHUNK // 16, grp, 0)

        fire_g(0, e0a, e1a, semGA)

        def pair(j2, _):
            jA = j2 * 2

            @pl.when(jA + 1 < it)
            def _():
                fire_g(jA + 1, e0b, e1b, semGB)
            waitg(jA, e0a, e1a, semGA)
            mkpid(e0a, e1a, apv0)
            gA = pltpu.async_copy(ones_v, acc_ap.at[apv0], semA, add=True)
            pltpu.sync_copy(ones_v, acc_deg.at[colix.at[jA]], add=True)

            @pl.when(jA + 2 < it)
            def _():
                fire_g(jA + 2, e0a, e1a, semGA)
            gA.wait()

            @pl.when(jA + 1 < it)
            def _():
                waitg(jA + 1, e0b, e1b, semGB)
                mkpid(e0b, e1b, apv1)
                gB = pltpu.async_copy(ones_v, acc_ap.at[apv1], semB, add=True)
                pltpu.sync_copy(ones_v, acc_deg.at[colix.at[jA + 1]], add=True)
                gB.wait()
            return 0
        lax.fori_loop(0, (it + 1) // 2, pair, 0)
        plsc.subcore_barrier()

        rw = NP // 16

        def wout(deg_h, ap_h):
            pltpu.sync_copy(acc_deg.at[pl.ds(s * rw, rw)],
                            zline.at[pl.ds(0, rw)])
            pltpu.sync_copy(zline.at[pl.ds(0, rw)],
                            deg_h.at[pl.ds(s * rw, rw)])

            def wr(j, _):
                pltpu.sync_copy(acc_ap.at[pl.ds(s * span + j * 4096, 4096)],
                                zline)
                pltpu.sync_copy(zline,
                                ap_h.at[pl.ds(s * span + j * 4096, 4096)])
                return 0
            lax.fori_loop(0, 9, wr, 0)
            pltpu.sync_copy(acc_ap.at[pl.ds(s * span + 9 * 4096, 3152)],
                            zline.at[pl.ds(0, 3152)])
            pltpu.sync_copy(zline.at[pl.ds(0, 3152)],
                            ap_h.at[pl.ds(s * span + 9 * 4096, 3152)])

        @pl.when(c == 0)
        def _():
            wout(deg0_h, ap0_h)

        @pl.when(c == 1)
        def _():
            wout(deg1_h, ap1_h)

    return k(row2, col2, bc_p)


def _edge_pipeline(hp_h, acc, rowix, colix, rows0, rows1, sem0, sem1, it):
    pltpu.async_copy(hp_h.at[rowix.at[0]], rows0, sem0)

    def pair(j2, _):
        jA = j2 * 2

        @pl.when(jA + 1 < it)
        def _():
            pltpu.async_copy(hp_h.at[rowix.at[jA + 1]], rows1, sem1)

        pltpu.make_async_copy(hp_h.at[rowix.at[jA]], rows0, sem0).wait()
        pltpu.sync_copy(rows0, acc.at[colix.at[jA]], add=True)

        @pl.when(jA + 2 < it)
        def _():
            pltpu.async_copy(hp_h.at[rowix.at[jA + 2]], rows0, sem0)

        @pl.when(jA + 1 < it)
        def _():
            pltpu.make_async_copy(hp_h.at[rowix.at[jA + 1]], rows1, sem1).wait()
            pltpu.sync_copy(rows1, acc.at[colix.at[jA + 1]], add=True)
        return 0
    lax.fori_loop(0, (it + 1) // 2, pair, 0)


def _sc_agg2(W, hp0, hp1, row2, col2):
    it = E // 16 // CHUNK
    BCH = 25
    NBLK = it // BCH

    @functools.partial(
        pl.kernel,
        out_type=(
            jax.ShapeDtypeStruct((NP, W), f32),
            jax.ShapeDtypeStruct((NP, W), f32),
        ),
        mesh=_mesh(),
        compiler_params=pltpu.CompilerParams(use_tc_tiling_on_sc=False),
        scratch_types=[
            pltpu.VMEM((BCH, CHUNK), i32),
            pltpu.VMEM((BCH, CHUNK), i32),
            pltpu.VMEM((BCH, CHUNK), i32),
            pltpu.VMEM((BCH, CHUNK), i32),
            pltpu.VMEM((CHUNK, W), f32),
            pltpu.VMEM((CHUNK, W), f32),
            pltpu.VMEM_SHARED((NP, W), f32),
            pltpu.SemaphoreType.DMA,
            pltpu.SemaphoreType.DMA,
            pltpu.SemaphoreType.DMA,
            pltpu.SemaphoreType.DMA,
        ],
    )
    def k(row_h, col_h, hp0_h, hp1_h, s0_h, s1_h,
          rixP, cixP, rixQ, cixQ, rows0, rows1, acc,
          sem0, sem1, semIP, semIQ):
        c = lax.axis_index("c")
        s = lax.axis_index("s")
        _zero2d(rows0, CHUNK, W)
        rpt = NP // 16

        def zb(i, _):
            pltpu.sync_copy(rows0, acc.at[pl.ds(s * rpt + i * CHUNK, CHUNK), :])
            return 0
        lax.fori_loop(0, rpt // CHUNK, zb, 0)
        plsc.subcore_barrier()

        def ldblk(b, rix, cix):
            pltpu.sync_copy(row_h.at[pl.ds(s * it + b * BCH, BCH), :], rix)
            pltpu.sync_copy(col_h.at[pl.ds(s * it + b * BCH, BCH), :], cix)

        def fireblk(b, rix, cix, semI):
            pltpu.async_copy(row_h.at[pl.ds(s * it + b * BCH, BCH), :], rix, semI)
            pltpu.async_copy(col_h.at[pl.ds(s * it + b * BCH, BCH), :], cix, semI)

        def waitblk(b, rix, cix, semI):
            pltpu.make_async_copy(row_h.at[pl.ds(s * it + b * BCH, BCH), :], rix, semI).wait()
            pltpu.make_async_copy(col_h.at[pl.ds(s * it + b * BCH, BCH), :], cix, semI).wait()

        def work(hp_h):
            ldblk(0, rixP, cixP)
            fireblk(1, rixQ, cixQ, semIQ)

            def bpair(k2, _):
                b = k2 * 2
                _edge_pipeline(hp_h, acc, rixP, cixP, rows0, rows1,
                               sem0, sem1, BCH)
                waitblk(b + 1, rixQ, cixQ, semIQ)

                @pl.when(b + 2 < NBLK)
                def _():
                    fireblk(b + 2, rixP, cixP, semIP)
                _edge_pipeline(hp_h, acc, rixQ, cixQ, rows0, rows1,
                               sem0, sem1, BCH)

                @pl.when(b + 2 < NBLK)
                def _():
                    waitblk(b + 2, rixP, cixP, semIP)

                    @pl.when(b + 3 < NBLK)
                    def _():
                        fireblk(b + 3, rixQ, cixQ, semIQ)
                return 0
            lax.fori_loop(0, NBLK // 2, bpair, 0)

        @pl.when(c == 0)
        def _():
            work(hp0_h)

        @pl.when(c == 1)
        def _():
            work(hp1_h)
        plsc.subcore_barrier()

        def wout(out_h):
            def w2(i2, _):
                r0 = s * rpt + i2 * 2 * CHUNK
                pltpu.sync_copy(acc.at[pl.ds(r0, CHUNK), :], rows0)
                d0 = pltpu.async_copy(rows0, out_h.at[pl.ds(r0, CHUNK), :], sem0)
                r1 = r0 + CHUNK
                pltpu.sync_copy(acc.at[pl.ds(r1, CHUNK), :], rows1)
                d1 = pltpu.async_copy(rows1, out_h.at[pl.ds(r1, CHUNK), :], sem1)
                d0.wait()
                d1.wait()
                return 0
            lax.fori_loop(0, rpt // CHUNK // 2, w2, 0)

        @pl.when(c == 0)
        def _():
            wout(s0_h)

        @pl.when(c == 1)
        def _():
            wout(s1_h)

    return k(row2, col2, hp0, hp1)


def _sc_agg_eb(W, hp, row2, col2):
    it = E // 32 // CHUNK

    @functools.partial(
        pl.kernel,
        out_type=(
            jax.ShapeDtypeStruct((NP, W), f32),
            jax.ShapeDtypeStruct((NP, W), f32),
        ),
        mesh=_mesh(),
        compiler_params=pltpu.CompilerParams(use_tc_tiling_on_sc=False),
        scratch_types=[
            pltpu.VMEM((E // 32 // CHUNK, CHUNK), i32),
            pltpu.VMEM((E // 32 // CHUNK, CHUNK), i32),
            pltpu.VMEM((CHUNK, W), f32),
            pltpu.VMEM((CHUNK, W), f32),
            pltpu.VMEM_SHARED((NP, W), f32),
            pltpu.SemaphoreType.DMA,
            pltpu.SemaphoreType.DMA,
        ],
    )
    def k(row_h, col_h, hp_h, s0_h, s1_h,
          rowix, colix, rows0, rows1, acc, sem0, sem1):
        c = lax.axis_index("c")
        s = lax.axis_index("s")
        tbase = (c * 16 + s) * it
        pltpu.sync_copy(row_h.at[pl.ds(tbase, it), :], rowix)
        pltpu.sync_copy(col_h.at[pl.ds(tbase, it), :], colix)
        _zero2d(rows0, CHUNK, W)
        rpt = NP // 16

        def zb(i, _):
            pltpu.sync_copy(rows0, acc.at[pl.ds(s * rpt + i * CHUNK, CHUNK), :])
            return 0
        lax.fori_loop(0, rpt // CHUNK, zb, 0)
        plsc.subcore_barrier()

        _edge_pipeline(hp_h, acc, rowix, colix, rows0, rows1,
                       sem0, sem1, it)
        plsc.subcore_barrier()

        def wout(out_h):
            def w2(i2, _):
                r0 = s * rpt + i2 * 2 * CHUNK
                pltpu.sync_copy(acc.at[pl.ds(r0, CHUNK), :], rows0)
                d0 = pltpu.async_copy(rows0, out_h.at[pl.ds(r0, CHUNK), :], sem0)
                r1 = r0 + CHUNK
                pltpu.sync_copy(acc.at[pl.ds(r1, CHUNK), :], rows1)
                d1 = pltpu.async_copy(rows1, out_h.at[pl.ds(r1, CHUNK), :], sem1)
                d0.wait()
                d1.wait()
                return 0
            lax.fori_loop(0, rpt // CHUNK // 2, w2, 0)

        @pl.when(c == 0)
        def _():
            wout(s0_h)

        @pl.when(c == 1)
        def _():
            wout(s1_h)

    return k(row2, col2, hp)


def _sc_pool(vals, idx_p, W):
    rpt = NP // 32

    @functools.partial(
        pl.kernel,
        out_type=(
            jax.ShapeDtypeStruct((MP, W), f32),
            jax.ShapeDtypeStruct((MP, W), f32),
        ),
        mesh=_mesh(),
        compiler_params=pltpu.CompilerParams(use_tc_tiling_on_sc=False),
        scratch_types=[
            pltpu.VMEM((CHUNK,), i32),
            pltpu.VMEM((CHUNK,), i32),
            pltpu.VMEM((CHUNK, W), f32),
            pltpu.VMEM((CHUNK, W), f32),
            pltpu.VMEM_SHARED((MP, W), f32),
            pltpu.SemaphoreType.DMA,
            pltpu.SemaphoreType.DMA,
        ],
    )
    def k(vals_h, idx_h, p0_h, p1_h, idx0, idx1, rows0, rows1, acc,
          sem0, sem1):
        c = lax.axis_index("c")
        s = lax.axis_index("s")
        mpt = MP // 16
        _zero2d(rows0, mpt, W)
        pltpu.sync_copy(rows0.at[pl.ds(0, mpt), :],
                        acc.at[pl.ds(s * mpt, mpt), :])
        plsc.subcore_barrier()

        base = (c * 16 + s) * rpt
        nch = rpt // CHUNK

        def fire(j, idxb, rows, sem):
            b = base + j * CHUNK
            pltpu.async_copy(idx_h.at[pl.ds(b, CHUNK)], idxb, sem)
            pltpu.async_copy(vals_h.at[pl.ds(b, CHUNK), :], rows, sem)

        def waitf(j, idxb, rows, sem):
            b = base + j * CHUNK
            pltpu.make_async_copy(idx_h.at[pl.ds(b, CHUNK)], idxb, sem).wait()
            pltpu.make_async_copy(vals_h.at[pl.ds(b, CHUNK), :], rows, sem).wait()

        fire(0, idx0, rows0, sem0)

        def pair(j2, _):
            j = j2 * 2

            @pl.when(j + 1 < nch)
            def _():
                fire(j + 1, idx1, rows1, sem1)
            waitf(j, idx0, rows0, sem0)
            pltpu.sync_copy(rows0, acc.at[idx0], add=True)

            @pl.when(j + 2 < nch)
            def _():
                fire(j + 2, idx0, rows0, sem0)

            @pl.when(j + 1 < nch)
            def _():
                waitf(j + 1, idx1, rows1, sem1)
                pltpu.sync_copy(rows1, acc.at[idx1], add=True)
            return 0
        lax.fori_loop(0, (nch + 1) // 2, pair, 0)
        plsc.subcore_barrier()

        def wout(out_h):
            pltpu.sync_copy(acc.at[pl.ds(s * mpt, mpt), :],
                            rows0.at[pl.ds(0, mpt), :])
            pltpu.sync_copy(rows0.at[pl.ds(0, mpt), :],
                            out_h.at[pl.ds(s * mpt, mpt), :])

        @pl.when(c == 0)
        def _():
            wout(p0_h)

        @pl.when(c == 1)
        def _():
            wout(p1_h)

    return k(vals, idx_p)


def _sc_gather(table, idx_p, W):
    rpt = NP // 32

    @functools.partial(
        pl.kernel,
        out_type=jax.ShapeDtypeStruct((NP, W), f32),
        mesh=_mesh(),
        compiler_params=pltpu.CompilerParams(use_tc_tiling_on_sc=False),
        scratch_types=[
            pltpu.VMEM((CHUNK,), i32),
            pltpu.VMEM((CHUNK,), i32),
            pltpu.VMEM((CHUNK, W), f32),
            pltpu.VMEM((CHUNK, W), f32),
            pltpu.SemaphoreType.DMA,
            pltpu.SemaphoreType.DMA,
        ],
    )
    def k(tab_h, idx_h, out_h, idx0, idx1, rows0, rows1, sem0, sem1):
        c = lax.axis_index("c")
        s = lax.axis_index("s")
        base = (c * 16 + s) * rpt
        nch = rpt // CHUNK

        pltpu.sync_copy(idx_h.at[pl.ds(base, CHUNK)], idx0)
        pltpu.async_copy(tab_h.at[idx0], rows0, sem0)

        def pair(j2, _):
            j = j2 * 2

            @pl.when(j + 1 < nch)
            def _():
                pltpu.sync_copy(idx_h.at[pl.ds(base + (j + 1) * CHUNK, CHUNK)],
                                idx1)
                pltpu.async_copy(tab_h.at[idx1], rows1, sem1)
            pltpu.make_async_copy(tab_h.at[idx0], rows0, sem0).wait()
            pltpu.sync_copy(rows0, out_h.at[pl.ds(base + j * CHUNK, CHUNK), :])

            @pl.when(j + 2 < nch)
            def _():
                pltpu.sync_copy(idx_h.at[pl.ds(base + (j + 2) * CHUNK, CHUNK)],
                                idx0)
                pltpu.async_copy(tab_h.at[idx0], rows0, sem0)

            @pl.when(j + 1 < nch)
            def _():
                pltpu.make_async_copy(tab_h.at[idx1], rows1, sem1).wait()
                pltpu.sync_copy(rows1,
                                out_h.at[pl.ds(base + (j + 1) * CHUNK, CHUNK), :])
            return 0
        lax.fori_loop(0, (nch + 1) // 2, pair, 0)

    return k(table, idx_p)


def _elu(v):
    return jnp.where(v > 0, v, jnp.exp(v) - 1.0)


def _dot(a, b):
    return jnp.dot(a, b, preferred_element_type=f32)


def _tc(fn, out_shape, *args):
    return pl.pallas_call(fn, out_shape=out_shape)(*args)


def _tc_pre(deg0, deg1, ap0_2d, ap1_2d):
    def body(d0, d1, a0, a1, dinv_r, pres_r, dp_r):
        deg = d0[...] + d1[...] + 1.0
        dinv_r[...] = lax.rsqrt(deg)
        pres = ((a0[...] + a1[...]) > 0).astype(f32)
        deg_p = jnp.sum(pres, axis=1, keepdims=True) + 1.0
        pres_r[...] = pres
        dp_r[...] = lax.rsqrt(deg_p)
    return _tc(body, (jax.ShapeDtypeStruct((NP,), f32),
                      jax.ShapeDtypeStruct((M, M), f32),
                      jax.ShapeDtypeStruct((M, 1), f32)),
               deg0, deg1, ap0_2d, ap1_2d)


def _tc0(x, Wg1, dinv_c):
    def body(x_r, w_r, dv_r, o0_r, o1_r):
        h = dv_r[...] * _dot(x_r[...], w_r[...])
        o0_r[...] = h[:, :32]
        o1_r[...] = h[:, 32:]
    return _tc(body, (jax.ShapeDtypeStruct((N, 32), f32),
                      jax.ShapeDtypeStruct((N, 32), f32)),
               x, Wg1, dinv_c)


def _tc1(s0, s1, hp0, hp1, dinv_c, bg1, fw1, fb1, Wg2):
    def body(s0_r, s1_r, h0_r, h1_r, dv_r, b_r, fw_r, fb_r, w2_r, o0_r, o1_r):
        S = jnp.concatenate([s0_r[...][:N], s1_r[...][:N]], axis=1)
        hp = jnp.concatenate([h0_r[...], h1_r[...]], axis=1)
        dv = dv_r[...]
        h1 = _elu(dv * (S + hp) + b_r[...])
        u1 = _elu(_dot(h1, fw_r[...]) + fb_r[...])
        h2p = dv * _dot(u1, w2_r[...])
        o0_r[...] = h2p[:, :128]
        o1_r[...] = h2p[:, 128:]
    return _tc(body, (jax.ShapeDtypeStruct((N, 128), f32),
                      jax.ShapeDtypeStruct((N, 128), f32)),
               s0, s1, hp0, hp1, dinv_c, bg1, fw1, fb1, Wg2)


def _tc2a(s0, s1, hp0, hp1, dinv_c, bg2, fw2, fb2):
    def body(s0_r, s1_r, h0_r, h1_r, dv_r, b_r, fw_r, fb_r, u_r):
        S = jnp.concatenate([s0_r[...][:N], s1_r[...][:N]], axis=1)
        hp = jnp.concatenate([h0_r[...], h1_r[...]], axis=1)
        h2 = _elu(dv_r[...] * (S + hp) + b_r[...])
        u_r[...] = _elu(_dot(h2, fw_r[...]) + fb_r[...])
    return _tc(body, jax.ShapeDtypeStruct((N, 256), f32),
               s0, s1, hp0, hp1, dinv_c, bg2, fw2, fb2)


def _tc2b(u2):
    def body(u_r, o_r):
        u = u_r[...]
        s1 = jnp.sum(u, axis=0, keepdims=True)
        s2 = jnp.sum(u * u, axis=0, keepdims=True)
        mu = s1 / N
        var = s2 / N - mu * mu
        y = (u - mu) * lax.rsqrt(var + 1e-5)
        aug = jnp.concatenate(
            [y, jnp.ones((N, 1), f32), jnp.zeros((N, 15), f32)], axis=1)
        o_r[...] = jnp.concatenate(
            [aug, jnp.zeros((NP - N, 272), f32)], axis=0)
    return _tc(body, jax.ShapeDtypeStruct((NP, 272), f32), u2)


def _tc3(ps0, ps1, y_aug, pres, dp, Wl1, bl1, fwl1, fbl1, Wl2, bl2,
         fwl2, fbl2, Wm1):
    def body(p0_r, p1_r, y_r, pr_r, dp_r, wl1_r, bl1_r, fw1_r, fb1_r,
             wl2_r, bl2_r, fw2_r, fb2_r, wm1_r, g_r, p_out_r, den_r):
        sums = p0_r[...][:M] + p1_r[...][:M]
        denom = jnp.maximum(sums[:, 256:257], 1.0)
        pooled = sums[:, :256] / denom
        pr = pr_r[...]
        dp = dp_r[...]

        def pconv(g, w, b):
            td = dp * _dot(g, w)
            return dp * (_dot(pr, td) + td) + b

        lx = _elu(pconv(pooled, wl1_r[...], bl1_r[...]))
        lx = _elu(_dot(lx, fw1_r[...]) + fb1_r[...])
        lx = _elu(pconv(lx, wl2_r[...], bl2_r[...]))
        lx = _elu(_dot(lx, fw2_r[...]) + fb2_r[...])
        wm1 = wm1_r[...]
        g_r[...] = _dot(lx, wm1[256:])
        p_out_r[...] = _dot(y_r[...][:N, :256], wm1[:256])
        den_r[...] = denom
    return _tc(body, (jax.ShapeDtypeStruct((M, 128), f32),
                      jax.ShapeDtypeStruct((N, 128), f32),
                      jax.ShapeDtypeStruct((M, 1), f32)),
               ps0, ps1, y_aug, pres, dp, Wl1, bl1, fwl1, fbl1,
               Wl2, bl2, fwl2, fbl2, Wm1)


def _tc4(gbc, p_mat, dinv_c):
    def body(g_r, p_r, dv_r, o0_r, o1_r):
        hp3 = dv_r[...] * (p_r[...] + g_r[...][:N])
        o0_r[...] = hp3[:, :64]
        o1_r[...] = hp3[:, 64:]
    return _tc(body, (jax.ShapeDtypeStruct((N, 64), f32),
                      jax.ShapeDtypeStruct((N, 64), f32)),
               gbc, p_mat, dinv_c)


def _tc5(s0, s1, hp0, hp1, dinv_c, bm1, fwm1, fbm1, Wm2):
    def body(s0_r, s1_r, h0_r, h1_r, dv_r, b_r, fw_r, fb_r, w2_r, o0_r, o1_r):
        S = jnp.concatenate([s0_r[...][:N], s1_r[...][:N]], axis=1)
        hp = jnp.concatenate([h0_r[...], h1_r[...]], axis=1)
        dv = dv_r[...]
        z1 = _elu(dv * (S + hp) + b_r[...])
        u = _elu(_dot(z1, fw_r[...]) + fb_r[...])
        hp4 = dv * _dot(u, w2_r[...])
        o0_r[...] = hp4[:, :16]
        o1_r[...] = hp4[:, 16:]
    return _tc(body, (jax.ShapeDtypeStruct((N, 16), f32),
                      jax.ShapeDtypeStruct((N, 16), f32)),
               s0, s1, hp0, hp1, dinv_c, bm1, fwm1, fbm1, Wm2)


def _tc6(s0, s1, hp0, hp1, dinv_c, bm2, fwm2, fbm2, W3p):
    def body(s0_r, s1_r, h0_r, h1_r, dv_r, b_r, fw_r, fb_r, w3_r, o_r):
        S = jnp.concatenate([s0_r[...][:N], s1_r[...][:N]], axis=1)
        hp = jnp.concatenate([h0_r[...], h1_r[...]], axis=1)
        dv = dv_r[...]
        h = _elu(dv * (S + hp) + b_r[...])
        u = _elu(_dot(h, fw_r[...]) + fb_r[...])
        o_r[...] = dv * _dot(u, w3_r[...])
    return _tc(body, jax.ShapeDtypeStruct((N, 16), f32),
               s0, s1, hp0, hp1, dinv_c, bm2, fwm2, fbm2, W3p)


def _tc7(s0, s1, hp5, dinv_c, b3p, fw3p, fb3p):
    def body(s0_r, s1_r, h_r, dv_r, b_r, fw_r, fb_r, o_r):
        S = s0_r[...][:N] + s1_r[...][:N]
        z = _elu(dv_r[...] * (S + h_r[...]) + b_r[...])
        z = _elu(_dot(z, fw_r[...]) + fb_r[...])
        o_r[...] = jnp.concatenate([z, jnp.zeros((NP - N, 16), f32)], axis=0)
    return _tc(body, jax.ShapeDtypeStruct((NP, 16), f32),
               s0, s1, hp5, dinv_c, b3p, fw3p, fb3p)


def _tc8(pz0, pz1, denom, pres, dp, WOp, bOp, fwOp, fbOp):
    def body(p0_r, p1_r, den_r, pr_r, dp_r, wo_r, bo_r, fw_r, fb_r, o_r):
        pooled = (p0_r[...][:M] + p1_r[...][:M]) / den_r[...]
        pr = pr_r[...]
        dp = dp_r[...]
        td = dp * _dot(pooled, wo_r[...])
        zz = _elu(dp * (_dot(pr, td) + td) + bo_r[...])
        o_r[...] = _elu(_dot(zz, fw_r[...]) + fb_r[...])
    return _tc(body, jax.ShapeDtypeStruct((M, 16), f32),
               pz0, pz1, denom, pres, dp, WOp, bOp, fwOp, fbOp)


def kernel(x, adj, num_graphs, in_batch, cluster, params):
    p = params
    row, col = adj[0], adj[1]

    W3p = jnp.zeros((32, 16), f32).at[:, :3].set(p["GCN_M3_W"])
    b3p = jnp.zeros((16,), f32).at[:3].set(p["GCN_M3_b"])
    fw3p = jnp.zeros((16, 16), f32).at[:3, :3].set(p["fc_M3_W"])
    fb3p = jnp.zeros((16,), f32).at[:3].set(p["fc_M3_b"])
    WOp = jnp.zeros((16, 16), f32).at[:3, :3].set(p["GCN_O1_W"])
    bOp = jnp.zeros((16,), f32).at[:3].set(p["GCN_O1_b"])
    fwOp = jnp.zeros((16, 16), f32).at[:3, :3].set(p["fc_O1_W"])
    fbOp = jnp.zeros((16,), f32).at[:3].set(p["fc_O1_b"])

    bc = cluster + in_batch * CN
    bc_p = jnp.concatenate([bc, jnp.zeros(NP - N, i32)])
    row2 = row.reshape(E // CHUNK, CHUNK)
    col2 = col.reshape(E // CHUNK, CHUNK)
    deg0, deg1, ap0, ap1 = _sc_prep(row2, col2, bc_p)
    ap0_2d = ap0[: M * M].reshape(M, M)
    ap1_2d = ap1[: M * M].reshape(M, M)
    dinv1d, pres, dp = _tc_pre(deg0, deg1, ap0_2d, ap1_2d)
    dinv_c = dinv1d[:N].reshape(N, 1)

    hp1a, hp1b = _tc0(x, p["GCN_G1_W"], dinv_c)
    s1a, s1b = _sc_agg2(32, hp1a, hp1b, row2, col2)
    hp2a, hp2b = _tc1(s1a, s1b, hp1a, hp1b, dinv_c, p["GCN_G1_b"],
                      p["fc_G1_W"], p["fc_G1_b"], p["GCN_G2_W"])
    s2a, s2b = _sc_agg2(128, hp2a, hp2b, row2, col2)
    u2 = _tc2a(s2a, s2b, hp2a, hp2b, dinv_c, p["GCN_G2_b"],
               p["fc_G2_W"], p["fc_G2_b"])
    y_aug = _tc2b(u2)

    ps0, ps1 = _sc_pool(y_aug, bc_p, 272)
    G, P, denom = _tc3(ps0, ps1, y_aug, pres, dp,
                       p["GCN_L1_W"], p["GCN_L1_b"], p["fc_L1_W"], p["fc_L1_b"],
                       p["GCN_L2_W"], p["GCN_L2_b"], p["fc_L2_W"], p["fc_L2_b"],
                       p["GCN_M1_W"])
    Gbc = _sc_gather(G, bc_p, 128)
    hp3a, hp3b = _tc4(Gbc, P, dinv_c)
    s3a, s3b = _sc_agg2(64, hp3a, hp3b, row2, col2)
    hp4a, hp4b = _tc5(s3a, s3b, hp3a, hp3b, dinv_c, p["GCN_M1_b"],
                      p["fc_M1_W"], p["fc_M1_b"], p["GCN_M2_W"])
    s4a, s4b = _sc_agg2(16, hp4a, hp4b, row2, col2)
    hp5 = _tc6(s4a, s4b, hp4a, hp4b, dinv_c, p["GCN_M2_b"],
               p["fc_M2_W"], p["fc_M2_b"], W3p)
    s5a, s5b = _sc_agg_eb(16, hp5, row2, col2)
    z8 = _tc7(s5a, s5b, hp5, dinv_c, b3p, fw3p, fb3p)

    pz0, pz1 = _sc_pool(z8, bc_p, 16)
    zz8 = _tc8(pz0, pz1, denom, pres, dp, WOp, bOp, fwOp, fbOp)
    orows = _sc_gather(zz8, bc_p, 16)
    return (orows[:N, :3], zz8[:, :3])
